# Initial kernel scaffold; baseline (speedup 1.0000x reference)
import jax
import jax.numpy as jnp
from jax.experimental import pallas as pl


def kernel(x, edge_index, batch, ligand_idx, additive_idx, base_idx, aryl_idx, W1, b1, W2, b2, E_lig, E_add, E_base, E_aryl, lin1_W, lin1_b, lin2_W, lin2_b):
    raise NotImplementedError("write your pallas kernel here")


# speedup vs baseline: 11.9655x; 11.9655x over previous
import functools

import jax
import jax.numpy as jnp
from jax import lax
from jax.experimental import pallas as pl
from jax.experimental.pallas import tpu as pltpu
from jax.experimental.pallas import tpu_sc as plsc

N = 50000
E = 800000
G = 512
F = 64
FH = 32
NC = 2
NS = 16
CH = 128

N_PAD = 51200
E_PAD = 802816
G_PAD = 640
NPT = N_PAD // NS
EPT_AGG = E_PAD // NS
EPT_DEG = E_PAD // (NC * NS)
GPT = G_PAD // NS

_mesh = plsc.VectorSubcoreMesh(
    core_axis_name="c", subcore_axis_name="s", num_cores=NC, num_subcores=NS)

_f32 = jnp.float32
_zeros16 = functools.partial(jnp.zeros, (16,), _f32)


def _zero_rows32(ref, nrows):
    def body(i, _):
        ref[i, pl.ds(0, 16)] = _zeros16()
        ref[i, pl.ds(16, 16)] = _zeros16()
        return 0
    lax.fori_loop(0, nrows, body, 0, unroll=2)


def _zero_rows128(ref, nrows):
    def body(i, _):
        for j in range(8):
            ref[i, pl.ds(j * 16, 16)] = _zeros16()
        return 0
    lax.fori_loop(0, nrows, body, 0)


@functools.partial(
    pl.kernel,
    out_type=jax.ShapeDtypeStruct((NC * NS, N_PAD // 128, 128), _f32),
    mesh=_mesh,
    compiler_params=pltpu.CompilerParams(needs_layout_passes=False, use_tc_tiling_on_sc=False),
    scratch_types=[
        pltpu.VMEM((CH,), jnp.int32),
        pltpu.VMEM((N_PAD // 128, 128), _f32),
    ],
)
def _sc_deg(dst_ref, out_ref, idx_v, cnt_v):
    c = lax.axis_index("c")
    s = lax.axis_index("s")
    wid = s * NC + c
    _zero_rows128(cnt_v, N_PAD // 128)
    ones = jnp.ones((16,), _f32)

    def body(k, _):
        base = wid * EPT_DEG + k * CH
        pltpu.sync_copy(dst_ref.at[pl.ds(base, CH)], idx_v)
        for j in range(CH // 16):
            iv = idx_v[pl.ds(j * 16, 16)]
            plsc.addupdate_scatter(
                cnt_v, [lax.shift_right_logical(iv, 7),
                        lax.bitwise_and(iv, 127)], ones)
        return 0

    lax.fori_loop(0, EPT_DEG // CH, body, 0)
    pltpu.sync_copy(cnt_v, out_ref.at[wid])


@functools.partial(
    pl.kernel,
    out_type=(
        jax.ShapeDtypeStruct((N_PAD, FH), _f32),
        jax.ShapeDtypeStruct((N_PAD, FH), _f32),
    ),
    mesh=_mesh,
    compiler_params=pltpu.CompilerParams(needs_layout_passes=False, use_tc_tiling_on_sc=False),
    scratch_types=[
        pltpu.VMEM((CH,), jnp.int32),
        pltpu.VMEM((CH,), jnp.int32),
        pltpu.VMEM((CH, FH), _f32),
        pltpu.VMEM((CH, FH), _f32),
        pltpu.VMEM_SHARED((N_PAD, FH), _f32),
        pltpu.SemaphoreType.DMA,
    ],
)
def _sc_agg(g0_ref, g1_ref, src_ref, dst_ref, a0_ref, a1_ref,
            src_v, dst_v, rows_v, zbuf_v, acc_sh, sem):
    c = lax.axis_index("c")
    s = lax.axis_index("s")
    _zero_rows32(zbuf_v, CH)

    def zacc(k, _):
        pltpu.sync_copy(zbuf_v, acc_sh.at[pl.ds(s * NPT + k * CH, CH)])
        return 0

    lax.fori_loop(0, NPT // CH, zacc, 0)
    plsc.subcore_barrier()

    def body(k, _):
        base = s * EPT_AGG + k * CH
        pltpu.sync_copy(src_ref.at[pl.ds(base, CH)], src_v)
        pltpu.sync_copy(dst_ref.at[pl.ds(base, CH)], dst_v)
        pl.when(c == 0)(
            lambda: pltpu.async_copy(g0_ref.at[src_v], rows_v, sem).wait())
        pl.when(c == 1)(
            lambda: pltpu.async_copy(g1_ref.at[src_v], rows_v, sem).wait())
        pltpu.sync_copy(rows_v, acc_sh.at[dst_v], add=True)
        return 0

    lax.fori_loop(0, EPT_AGG // CH, body, 0)
    plsc.subcore_barrier()
    sl = pl.ds(s * NPT, NPT)
    pl.when(c == 0)(lambda: pltpu.sync_copy(acc_sh.at[sl], a0_ref.at[sl]))
    pl.when(c == 1)(lambda: pltpu.sync_copy(acc_sh.at[sl], a1_ref.at[sl]))


@functools.partial(
    pl.kernel,
    out_type=(
        jax.ShapeDtypeStruct((G_PAD, FH), _f32),
        jax.ShapeDtypeStruct((G_PAD, FH), _f32),
        jax.ShapeDtypeStruct((NS, G_PAD // 128, 128), _f32),
    ),
    mesh=_mesh,
    compiler_params=pltpu.CompilerParams(needs_layout_passes=False, use_tc_tiling_on_sc=False),
    scratch_types=[
        pltpu.VMEM((CH,), jnp.int32),
        pltpu.VMEM((CH, FH), _f32),
        pltpu.VMEM((CH, FH), _f32),
        pltpu.VMEM((G_PAD // 128, 128), _f32),
        pltpu.VMEM_SHARED((G_PAD, FH), _f32),
    ],
)
def _sc_pool(h0_ref, h1_ref, batch_ref, p0_ref, p1_ref, cnt_ref,
             idx_v, rows_v, zbuf_v, cnt_v, acc_sh):
    c = lax.axis_index("c")
    s = lax.axis_index("s")
    _zero_rows32(zbuf_v, CH)
    _zero_rows128(cnt_v, G_PAD // 128)
    pltpu.sync_copy(zbuf_v.at[pl.ds(0, GPT)], acc_sh.at[pl.ds(s * GPT, GPT)])
    plsc.subcore_barrier()
    ones = jnp.ones((16,), _f32)

    def body(k, _):
        base = s * NPT + k * CH
        pltpu.sync_copy(batch_ref.at[pl.ds(base, CH)], idx_v)
        pl.when(c == 0)(
            lambda: pltpu.sync_copy(h0_ref.at[pl.ds(base, CH)], rows_v))
        pl.when(c == 1)(
            lambda: pltpu.sync_copy(h1_ref.at[pl.ds(base, CH)], rows_v))
        pltpu.sync_copy(rows_v, acc_sh.at[idx_v], add=True)

        def count():
            for j in range(CH // 16):
                iv = idx_v[pl.ds(j * 16, 16)]
                plsc.addupdate_scatter(
                    cnt_v, [lax.shift_right_logical(iv, 7),
                            lax.bitwise_and(iv, 127)], ones)
        pl.when(c == 0)(count)
        return 0

    lax.fori_loop(0, NPT // CH, body, 0)
    plsc.subcore_barrier()
    sl = pl.ds(s * GPT, GPT)
    pl.when(c == 0)(lambda: pltpu.sync_copy(acc_sh.at[sl], p0_ref.at[sl]))
    pl.when(c == 1)(lambda: pltpu.sync_copy(acc_sh.at[sl], p1_ref.at[sl]))
    pl.when(c == 0)(lambda: pltpu.sync_copy(cnt_v, cnt_ref.at[s]))


_BN = 1024


def _tc_a_body(x_ref, degp_ref, w1_ref, g0_ref, g1_ref, dinv_ref):
    deg = jnp.sum(degp_ref[...], axis=0)
    dinv = lax.rsqrt(deg + 1.0)
    hw = jnp.dot(x_ref[...], w1_ref[...], preferred_element_type=_f32)
    g = hw * dinv[:, None]
    g0_ref[...] = g[:, :FH]
    g1_ref[...] = g[:, FH:]
    dinv_ref[...] = dinv[:, None]


def _tc_a(xp, degp, W1):
    grid = (N_PAD // _BN,)
    return pl.pallas_call(
        _tc_a_body,
        grid=grid,
        in_specs=[
            pl.BlockSpec((_BN, F), lambda i: (i, 0)),
            pl.BlockSpec((NC * NS, _BN), lambda i: (0, i)),
            pl.BlockSpec((F, F), lambda i: (0, 0)),
        ],
        out_specs=(
            pl.BlockSpec((_BN, FH), lambda i: (i, 0)),
            pl.BlockSpec((_BN, FH), lambda i: (i, 0)),
            pl.BlockSpec((_BN, 1), lambda i: (i, 0)),
        ),
        out_shape=(
            jax.ShapeDtypeStruct((N_PAD, FH), _f32),
            jax.ShapeDtypeStruct((N_PAD, FH), _f32),
            jax.ShapeDtypeStruct((N_PAD, 1), _f32),
        ),
    )(xp, degp, W1)


def _tc_mid_body(a0_ref, a1_ref, g0_ref, g1_ref, dinv_ref, w_ref, b_ref,
                 o0_ref, o1_ref):
    dinv = dinv_ref[...]
    hfull = jnp.concatenate(
        [a0_ref[...] + g0_ref[...], a1_ref[...] + g1_ref[...]], axis=1)
    h = jnp.maximum(hfull * dinv + b_ref[...], 0.0)
    hw = jnp.dot(h, w_ref[...], preferred_element_type=_f32)
    g = hw * dinv
    o0_ref[...] = g[:, :FH]
    o1_ref[...] = g[:, FH:]


def _tc_mid(a0, a1, g0, g1, dinv, W2, b1):
    grid = (N_PAD // _BN,)
    nspec = pl.BlockSpec((_BN, FH), lambda i: (i, 0))
    return pl.pallas_call(
        _tc_mid_body,
        grid=grid,
        in_specs=[
            nspec, nspec, nspec, nspec,
            pl.BlockSpec((_BN, 1), lambda i: (i, 0)),
            pl.BlockSpec((F, F), lambda i: (0, 0)),
            pl.BlockSpec((1, F), lambda i: (0, 0)),
        ],
        out_specs=(nspec, nspec),
        out_shape=(
            jax.ShapeDtypeStruct((N_PAD, FH), _f32),
            jax.ShapeDtypeStruct((N_PAD, FH), _f32),
        ),
    )(a0, a1, g0, g1, dinv, W2, b1)


def _tc_last_body(a0_ref, a1_ref, g0_ref, g1_ref, dinv_ref, b_ref,
                  o0_ref, o1_ref):
    dinv = dinv_ref[...]
    b = b_ref[...]
    o0_ref[...] = jnp.maximum(
        (a0_ref[...] + g0_ref[...]) * dinv + b[:, :FH], 0.0)
    o1_ref[...] = jnp.maximum(
        (a1_ref[...] + g1_ref[...]) * dinv + b[:, FH:], 0.0)


def _tc_last(a0, a1, g0, g1, dinv, b2):
    grid = (N_PAD // _BN,)
    nspec = pl.BlockSpec((_BN, FH), lambda i: (i, 0))
    return pl.pallas_call(
        _tc_last_body,
        grid=grid,
        in_specs=[
            nspec, nspec, nspec, nspec,
            pl.BlockSpec((_BN, 1), lambda i: (i, 0)),
            pl.BlockSpec((1, F), lambda i: (0, 0)),
        ],
        out_specs=(nspec, nspec),
        out_shape=(
            jax.ShapeDtypeStruct((N_PAD, FH), _f32),
            jax.ShapeDtypeStruct((N_PAD, FH), _f32),
        ),
    )(a0, a1, g0, g1, dinv, b2)


def _tc_head_body(p0_ref, p1_ref, cntp_ref, lig_ref, add_ref, bas_ref,
                  ary_ref, el_ref, ea_ref, eb_ref, ey_ref, w1_ref, b1_ref,
                  w2_ref, b2_ref, out_ref):
    cnt = jnp.sum(cntp_ref[...], axis=0)[:G]
    psum = jnp.concatenate([p0_ref[...], p1_ref[...]], axis=1)[:G]
    pooled = psum / jnp.maximum(cnt, 1.0)[:, None]

    w1 = w1_ref[...]
    z = jnp.dot(pooled, w1[:F], preferred_element_type=_f32)

    def emb(idx_ref, table_ref, row0, nrows):
        k = table_ref.shape[0]
        oh = (idx_ref[...] ==
              lax.broadcasted_iota(jnp.int32, (G, k), 1)).astype(_f32)
        tw = jnp.dot(table_ref[...], w1[row0:row0 + nrows],
                     preferred_element_type=_f32)
        return jnp.dot(oh, tw, preferred_element_type=_f32)

    EMB = 16
    z = z + emb(lig_ref, el_ref, F, EMB)
    z = z + emb(add_ref, ea_ref, F + EMB, EMB)
    z = z + emb(bas_ref, eb_ref, F + 2 * EMB, EMB)
    z = z + emb(ary_ref, ey_ref, F + 3 * EMB, EMB)
    z = jnp.maximum(z + b1_ref[...], 0.0)
    out_ref[...] = (jnp.dot(z, w2_ref[...], preferred_element_type=_f32)
                    + b2_ref[...])


def _tc_head(p0, p1, cntp, lig, add, bas, ary, E_lig, E_add, E_base, E_aryl,
             lin1_W, lin1_b, lin2_W, lin2_b):
    args = (p0, p1, cntp, lig, add, bas, ary, E_lig, E_add, E_base, E_aryl,
            lin1_W, lin1_b, lin2_W, lin2_b)

    def spec(a):
        nd = a.ndim
        return pl.BlockSpec(a.shape, lambda: (0,) * nd)

    return pl.pallas_call(
        _tc_head_body,
        in_specs=[spec(a) for a in args],
        out_specs=pl.BlockSpec((G, 1), lambda: (0, 0)),
        out_shape=jax.ShapeDtypeStruct((G, 1), _f32),
    )(*args)


def kernel(x, edge_index, batch, ligand_idx, additive_idx, base_idx, aryl_idx,
           W1, b1, W2, b2, E_lig, E_add, E_base, E_aryl,
           lin1_W, lin1_b, lin2_W, lin2_b):
    xp = jnp.pad(x, ((0, N_PAD - N), (0, 0)))
    srcp = jnp.pad(edge_index[0], (0, E_PAD - E), constant_values=N_PAD - 1)
    dstp = jnp.pad(edge_index[1], (0, E_PAD - E), constant_values=N_PAD - 1)
    batchp = jnp.pad(batch, (0, N_PAD - N), constant_values=G)

    degp = _sc_deg(dstp).reshape(NC * NS, N_PAD)
    g0, g1, dinv = _tc_a(xp, degp, W1)
    a0, a1 = _sc_agg(g0, g1, srcp, dstp)
    g20, g21 = _tc_mid(a0, a1, g0, g1, dinv, W2, b1.reshape(1, F))
    a20, a21 = _sc_agg(g20, g21, srcp, dstp)
    h0, h1 = _tc_last(a20, a21, g20, g21, dinv, b2.reshape(1, F))
    p0, p1, cntp = _sc_pool(h0, h1, batchp)
    out = _tc_head(
        p0, p1, cntp.reshape(NS, G_PAD),
        ligand_idx.reshape(G, 1), additive_idx.reshape(G, 1),
        base_idx.reshape(G, 1), aryl_idx.reshape(G, 1),
        E_lig, E_add, E_base, E_aryl,
        lin1_W, lin1_b.reshape(1, F), lin2_W, lin2_b.reshape(1, 1))
    return out


# speedup vs baseline: 17.7256x; 1.4814x over previous
import functools

import jax
import jax.numpy as jnp
from jax import lax
from jax.experimental import pallas as pl
from jax.experimental.pallas import tpu as pltpu
from jax.experimental.pallas import tpu_sc as plsc

N = 50000
E = 800000
G = 512
F = 64
FH = 32
NC = 2
NS = 16
CH = 128

N_PAD = 51200
E_PAD = 819200
G_PAD = 640
NPT = N_PAD // NS
NCHUNK = E_PAD // CH
SUP = 4
CPT_AGG = NCHUNK // NS
NSUPER = CPT_AGG // SUP
CPT_DEG = NCHUNK // (NC * NS)
GPT = G_PAD // NS

_mesh = plsc.VectorSubcoreMesh(
    core_axis_name="c", subcore_axis_name="s", num_cores=NC, num_subcores=NS)

_f32 = jnp.float32
_zeros16 = functools.partial(jnp.zeros, (16,), _f32)


def _zero_rows32(ref, nrows):
    def body(i, _):
        ref[i, pl.ds(0, 16)] = _zeros16()
        ref[i, pl.ds(16, 16)] = _zeros16()
        return 0
    lax.fori_loop(0, nrows, body, 0, unroll=2)


def _zero_rows128(ref, nrows):
    def body(i, _):
        for j in range(8):
            ref[i, pl.ds(j * 16, 16)] = _zeros16()
        return 0
    lax.fori_loop(0, nrows, body, 0)


@functools.partial(
    pl.kernel,
    out_type=jax.ShapeDtypeStruct((NC * NS, N_PAD // 128, 128), _f32),
    mesh=_mesh,
    compiler_params=pltpu.CompilerParams(needs_layout_passes=False, use_tc_tiling_on_sc=False),
    scratch_types=[
        pltpu.VMEM((SUP * CH,), jnp.int32),
        pltpu.VMEM((N_PAD // 128, 128), _f32),
    ],
)
def _sc_deg(dst_ref, out_ref, idx_v, cnt_v):
    c = lax.axis_index("c")
    s = lax.axis_index("s")
    wid = s * NC + c
    _zero_rows128(cnt_v, N_PAD // 128)
    ones = jnp.ones((16,), _f32)

    def body(k, _):
        base = (wid * CPT_DEG + k * SUP) * CH
        pltpu.sync_copy(dst_ref.at[pl.ds(base, SUP * CH)], idx_v)
        for q in range(SUP):
            for j in range(CH // 16):
                iv = idx_v[pl.ds(q * CH + j * 16, 16)]
                plsc.addupdate_scatter(
                    cnt_v, [lax.shift_right_logical(iv, 7),
                            lax.bitwise_and(iv, 127)], ones)
        return 0

    lax.fori_loop(0, CPT_DEG // SUP, body, 0)
    pltpu.sync_copy(cnt_v, out_ref.at[wid])


@functools.partial(
    pl.kernel,
    out_type=(
        jax.ShapeDtypeStruct((N_PAD, FH), _f32),
        jax.ShapeDtypeStruct((N_PAD, FH), _f32),
    ),
    mesh=_mesh,
    compiler_params=pltpu.CompilerParams(needs_layout_passes=False, use_tc_tiling_on_sc=False),
    scratch_types=[
        pltpu.VMEM((4, SUP * CH), jnp.int32),
        pltpu.VMEM((4, SUP, CH), jnp.int32),
        pltpu.VMEM((SUP, CH, FH), _f32),
        pltpu.VMEM_SHARED((N_PAD, FH), _f32),
        pltpu.SemaphoreType.DMA,
        pltpu.SemaphoreType.DMA,
        pltpu.SemaphoreType.DMA,
        pltpu.SemaphoreType.DMA,
        pltpu.SemaphoreType.DMA,
        pltpu.SemaphoreType.DMA,
    ],
)
def _sc_agg(g0_ref, g1_ref, src_ref, dst_ref, a0_ref, a1_ref,
            src_v, dst_v, rows_v, acc_sh,
            sem_i0, sem_i1, sem_i2, sem_i3, sem_g, sem_s):
    c = lax.axis_index("c")
    s = lax.axis_index("s")
    sem_i = (sem_i0, sem_i1, sem_i2, sem_i3)

    def zrow(i, _):
        rows_v[0, i, pl.ds(0, 16)] = _zeros16()
        rows_v[0, i, pl.ds(16, 16)] = _zeros16()
        return 0

    lax.fori_loop(0, CH, zrow, 0, unroll=2)

    def zacc(k, _):
        pltpu.sync_copy(rows_v.at[0], acc_sh.at[pl.ds(s * NPT + k * CH, CH)])
        return 0

    lax.fori_loop(0, NPT // CH, zacc, 0)
    plsc.subcore_barrier()

    cbase = s * CPT_AGG

    def fire_idx(sup, q):
        ebase = (cbase + sup * SUP) * CH
        pltpu.async_copy(
            src_ref.at[pl.ds(ebase, SUP * CH)], src_v.at[q], sem_i[q])
        for j in range(SUP):
            pltpu.async_copy(
                dst_ref.at[pl.ds(ebase + j * CH, CH)], dst_v.at[q, j],
                sem_i[q])

    def wait_idx(q):
        pltpu.make_async_copy(
            src_ref.at[pl.ds(0, SUP * CH)], src_v.at[q], sem_i[q]).wait()
        for j in range(SUP):
            pltpu.make_async_copy(
                dst_ref.at[pl.ds(0, CH)], dst_v.at[q, j], sem_i[q]).wait()

    def drain_scatters(q):
        for j in range(SUP):
            pltpu.make_async_copy(
                rows_v.at[j], acc_sh.at[dst_v.at[q, j]], sem_s).wait()

    def step(i, q):
        S = 4 * i + q
        wait_idx(q)
        if q > 0:
            drain_scatters((q + 3) % 4)
        else:
            pl.when(i > 0)(lambda: drain_scatters(3))
        pl.when(S + 2 < NSUPER)(lambda: fire_idx(S + 2, (q + 2) % 4))

        def fire_gathers_0():
            for j in range(SUP):
                pltpu.async_copy(
                    g0_ref.at[src_v.at[q, pl.ds(j * CH, CH)]], rows_v.at[j],
                    sem_g)

        def fire_gathers_1():
            for j in range(SUP):
                pltpu.async_copy(
                    g1_ref.at[src_v.at[q, pl.ds(j * CH, CH)]], rows_v.at[j],
                    sem_g)

        pl.when(c == 0)(fire_gathers_0)
        pl.when(c == 1)(fire_gathers_1)
        for j in range(SUP):
            pltpu.make_async_copy(
                g0_ref.at[src_v.at[q, pl.ds(j * CH, CH)]], rows_v.at[j],
                sem_g).wait()
        for j in range(SUP):
            pltpu.async_copy(
                rows_v.at[j], acc_sh.at[dst_v.at[q, j]], sem_s, add=True)

    fire_idx(0, 0)
    fire_idx(1, 1)

    def outer(i, _):
        step(i, 0)
        step(i, 1)
        step(i, 2)
        step(i, 3)
        return 0

    lax.fori_loop(0, NSUPER // 4, outer, 0)
    drain_scatters(3)
    plsc.subcore_barrier()
    sl = pl.ds(s * NPT, NPT)
    pl.when(c == 0)(lambda: pltpu.sync_copy(acc_sh.at[sl], a0_ref.at[sl]))
    pl.when(c == 1)(lambda: pltpu.sync_copy(acc_sh.at[sl], a1_ref.at[sl]))


@functools.partial(
    pl.kernel,
    out_type=(
        jax.ShapeDtypeStruct((G_PAD, FH), _f32),
        jax.ShapeDtypeStruct((G_PAD, FH), _f32),
        jax.ShapeDtypeStruct((NS, G_PAD // 128, 128), _f32),
    ),
    mesh=_mesh,
    compiler_params=pltpu.CompilerParams(needs_layout_passes=False, use_tc_tiling_on_sc=False),
    scratch_types=[
        pltpu.VMEM((CH,), jnp.int32),
        pltpu.VMEM((CH, FH), _f32),
        pltpu.VMEM((CH, FH), _f32),
        pltpu.VMEM((G_PAD // 128, 128), _f32),
        pltpu.VMEM_SHARED((G_PAD, FH), _f32),
    ],
)
def _sc_pool(h0_ref, h1_ref, batch_ref, p0_ref, p1_ref, cnt_ref,
             idx_v, rows_v, zbuf_v, cnt_v, acc_sh):
    c = lax.axis_index("c")
    s = lax.axis_index("s")
    _zero_rows32(zbuf_v, CH)
    _zero_rows128(cnt_v, G_PAD // 128)
    pltpu.sync_copy(zbuf_v.at[pl.ds(0, GPT)], acc_sh.at[pl.ds(s * GPT, GPT)])
    plsc.subcore_barrier()
    ones = jnp.ones((16,), _f32)

    def body(k, _):
        base = s * NPT + k * CH
        pltpu.sync_copy(batch_ref.at[pl.ds(base, CH)], idx_v)
        pl.when(c == 0)(
            lambda: pltpu.sync_copy(h0_ref.at[pl.ds(base, CH)], rows_v))
        pl.when(c == 1)(
            lambda: pltpu.sync_copy(h1_ref.at[pl.ds(base, CH)], rows_v))
        pltpu.sync_copy(rows_v, acc_sh.at[idx_v], add=True)

        def count():
            for j in range(CH // 16):
                iv = idx_v[pl.ds(j * 16, 16)]
                plsc.addupdate_scatter(
                    cnt_v, [lax.shift_right_logical(iv, 7),
                            lax.bitwise_and(iv, 127)], ones)
        pl.when(c == 0)(count)
        return 0

    lax.fori_loop(0, NPT // CH, body, 0)
    plsc.subcore_barrier()
    sl = pl.ds(s * GPT, GPT)
    pl.when(c == 0)(lambda: pltpu.sync_copy(acc_sh.at[sl], p0_ref.at[sl]))
    pl.when(c == 1)(lambda: pltpu.sync_copy(acc_sh.at[sl], p1_ref.at[sl]))
    pl.when(c == 0)(lambda: pltpu.sync_copy(cnt_v, cnt_ref.at[s]))


_BN = 1024


def _tc_a_body(x_ref, degp_ref, w1_ref, g0_ref, g1_ref, dinv_ref):
    deg = jnp.sum(degp_ref[...], axis=0)
    dinv = lax.rsqrt(deg + 1.0)
    hw = jnp.dot(x_ref[...], w1_ref[...], preferred_element_type=_f32)
    g = hw * dinv[:, None]
    g0_ref[...] = g[:, :FH]
    g1_ref[...] = g[:, FH:]
    dinv_ref[...] = dinv[:, None]


def _tc_a(xp, degp, W1):
    grid = (N_PAD // _BN,)
    return pl.pallas_call(
        _tc_a_body,
        grid=grid,
        in_specs=[
            pl.BlockSpec((_BN, F), lambda i: (i, 0)),
            pl.BlockSpec((NC * NS, _BN), lambda i: (0, i)),
            pl.BlockSpec((F, F), lambda i: (0, 0)),
        ],
        out_specs=(
            pl.BlockSpec((_BN, FH), lambda i: (i, 0)),
            pl.BlockSpec((_BN, FH), lambda i: (i, 0)),
            pl.BlockSpec((_BN, 1), lambda i: (i, 0)),
        ),
        out_shape=(
            jax.ShapeDtypeStruct((N_PAD, FH), _f32),
            jax.ShapeDtypeStruct((N_PAD, FH), _f32),
            jax.ShapeDtypeStruct((N_PAD, 1), _f32),
        ),
    )(xp, degp, W1)


def _tc_mid_body(a0_ref, a1_ref, g0_ref, g1_ref, dinv_ref, w_ref, b_ref,
                 o0_ref, o1_ref):
    dinv = dinv_ref[...]
    hfull = jnp.concatenate(
        [a0_ref[...] + g0_ref[...], a1_ref[...] + g1_ref[...]], axis=1)
    h = jnp.maximum(hfull * dinv + b_ref[...], 0.0)
    hw = jnp.dot(h, w_ref[...], preferred_element_type=_f32)
    g = hw * dinv
    o0_ref[...] = g[:, :FH]
    o1_ref[...] = g[:, FH:]


def _tc_mid(a0, a1, g0, g1, dinv, W2, b1):
    grid = (N_PAD // _BN,)
    nspec = pl.BlockSpec((_BN, FH), lambda i: (i, 0))
    return pl.pallas_call(
        _tc_mid_body,
        grid=grid,
        in_specs=[
            nspec, nspec, nspec, nspec,
            pl.BlockSpec((_BN, 1), lambda i: (i, 0)),
            pl.BlockSpec((F, F), lambda i: (0, 0)),
            pl.BlockSpec((1, F), lambda i: (0, 0)),
        ],
        out_specs=(nspec, nspec),
        out_shape=(
            jax.ShapeDtypeStruct((N_PAD, FH), _f32),
            jax.ShapeDtypeStruct((N_PAD, FH), _f32),
        ),
    )(a0, a1, g0, g1, dinv, W2, b1)


def _tc_last_body(a0_ref, a1_ref, g0_ref, g1_ref, dinv_ref, b_ref,
                  o0_ref, o1_ref):
    dinv = dinv_ref[...]
    b = b_ref[...]
    o0_ref[...] = jnp.maximum(
        (a0_ref[...] + g0_ref[...]) * dinv + b[:, :FH], 0.0)
    o1_ref[...] = jnp.maximum(
        (a1_ref[...] + g1_ref[...]) * dinv + b[:, FH:], 0.0)


def _tc_last(a0, a1, g0, g1, dinv, b2):
    grid = (N_PAD // _BN,)
    nspec = pl.BlockSpec((_BN, FH), lambda i: (i, 0))
    return pl.pallas_call(
        _tc_last_body,
        grid=grid,
        in_specs=[
            nspec, nspec, nspec, nspec,
            pl.BlockSpec((_BN, 1), lambda i: (i, 0)),
            pl.BlockSpec((1, F), lambda i: (0, 0)),
        ],
        out_specs=(nspec, nspec),
        out_shape=(
            jax.ShapeDtypeStruct((N_PAD, FH), _f32),
            jax.ShapeDtypeStruct((N_PAD, FH), _f32),
        ),
    )(a0, a1, g0, g1, dinv, b2)


def _tc_head_body(p0_ref, p1_ref, cntp_ref, lig_ref, add_ref, bas_ref,
                  ary_ref, el_ref, ea_ref, eb_ref, ey_ref, w1_ref, b1_ref,
                  w2_ref, b2_ref, out_ref):
    cnt = jnp.sum(cntp_ref[...], axis=0)[:G]
    psum = jnp.concatenate([p0_ref[...], p1_ref[...]], axis=1)[:G]
    pooled = psum / jnp.maximum(cnt, 1.0)[:, None]

    w1 = w1_ref[...]
    z = jnp.dot(pooled, w1[:F], preferred_element_type=_f32)

    def emb(idx_ref, table_ref, row0, nrows):
        k = table_ref.shape[0]
        oh = (idx_ref[...] ==
              lax.broadcasted_iota(jnp.int32, (G, k), 1)).astype(_f32)
        tw = jnp.dot(table_ref[...], w1[row0:row0 + nrows],
                     preferred_element_type=_f32)
        return jnp.dot(oh, tw, preferred_element_type=_f32)

    EMB = 16
    z = z + emb(lig_ref, el_ref, F, EMB)
    z = z + emb(add_ref, ea_ref, F + EMB, EMB)
    z = z + emb(bas_ref, eb_ref, F + 2 * EMB, EMB)
    z = z + emb(ary_ref, ey_ref, F + 3 * EMB, EMB)
    z = jnp.maximum(z + b1_ref[...], 0.0)
    out_ref[...] = (jnp.dot(z, w2_ref[...], preferred_element_type=_f32)
                    + b2_ref[...])


def _tc_head(p0, p1, cntp, lig, add, bas, ary, E_lig, E_add, E_base, E_aryl,
             lin1_W, lin1_b, lin2_W, lin2_b):
    args = (p0, p1, cntp, lig, add, bas, ary, E_lig, E_add, E_base, E_aryl,
            lin1_W, lin1_b, lin2_W, lin2_b)

    def spec(a):
        nd = a.ndim
        return pl.BlockSpec(a.shape, lambda: (0,) * nd)

    return pl.pallas_call(
        _tc_head_body,
        in_specs=[spec(a) for a in args],
        out_specs=pl.BlockSpec((G, 1), lambda: (0, 0)),
        out_shape=jax.ShapeDtypeStruct((G, 1), _f32),
    )(*args)


def kernel(x, edge_index, batch, ligand_idx, additive_idx, base_idx, aryl_idx,
           W1, b1, W2, b2, E_lig, E_add, E_base, E_aryl,
           lin1_W, lin1_b, lin2_W, lin2_b):
    xp = jnp.pad(x, ((0, N_PAD - N), (0, 0)))
    ep = jnp.pad(edge_index, ((0, 0), (0, E_PAD - E)),
                 constant_values=N_PAD - 1)
    srcp = ep[0]
    dstp = ep[1]
    batchp = jnp.pad(batch, (0, N_PAD - N), constant_values=G)

    degp = _sc_deg(dstp).reshape(NC * NS, N_PAD)
    g0, g1, dinv = _tc_a(xp, degp, W1)
    a0, a1 = _sc_agg(g0, g1, srcp, dstp)
    g20, g21 = _tc_mid(a0, a1, g0, g1, dinv, W2, b1.reshape(1, F))
    a20, a21 = _sc_agg(g20, g21, srcp, dstp)
    h0, h1 = _tc_last(a20, a21, g20, g21, dinv, b2.reshape(1, F))
    p0, p1, cntp = _sc_pool(h0, h1, batchp)
    out = _tc_head(
        p0, p1, cntp.reshape(NS, G_PAD),
        ligand_idx.reshape(G, 1), additive_idx.reshape(G, 1),
        base_idx.reshape(G, 1), aryl_idx.reshape(G, 1),
        E_lig, E_add, E_base, E_aryl,
        lin1_W, lin1_b.reshape(1, F), lin2_W, lin2_b.reshape(1, 1))
    return out


# speedup vs baseline: 20.0786x; 1.1327x over previous
import functools

import jax
import jax.numpy as jnp
from jax import lax
from jax.experimental import pallas as pl
from jax.experimental.pallas import tpu as pltpu
from jax.experimental.pallas import tpu_sc as plsc

N = 50000
E = 800000
G = 512
F = 64
FH = 32
NC = 2
NS = 16
CH = 128

N_PAD = 51200
E_PAD = 819200
G_PAD = 640
NPT = N_PAD // NS
NCHUNK = E_PAD // CH
SUP = 4
RING = 5
CPT_AGG = NCHUNK // NS
NSUPER = CPT_AGG // SUP
CPT_DEG = NCHUNK // (NC * NS)
GPT = G_PAD // NS

_mesh = plsc.VectorSubcoreMesh(
    core_axis_name="c", subcore_axis_name="s", num_cores=NC, num_subcores=NS)

_f32 = jnp.float32
_zeros16 = functools.partial(jnp.zeros, (16,), _f32)


def _zero_rows32(ref, nrows):
    def body(i, _):
        ref[i, pl.ds(0, 16)] = _zeros16()
        ref[i, pl.ds(16, 16)] = _zeros16()
        return 0
    lax.fori_loop(0, nrows, body, 0, unroll=2)


def _zero_rows128(ref, nrows):
    def body(i, _):
        for j in range(8):
            ref[i, pl.ds(j * 16, 16)] = _zeros16()
        return 0
    lax.fori_loop(0, nrows, body, 0)


@functools.partial(
    pl.kernel,
    out_type=jax.ShapeDtypeStruct((NC * NS, N_PAD // 128, 128), _f32),
    mesh=_mesh,
    compiler_params=pltpu.CompilerParams(needs_layout_passes=False, use_tc_tiling_on_sc=False),
    scratch_types=[
        pltpu.VMEM((SUP * CH,), jnp.int32),
        pltpu.VMEM((N_PAD // 128, 128), _f32),
    ],
)
def _sc_deg(dst_ref, out_ref, idx_v, cnt_v):
    c = lax.axis_index("c")
    s = lax.axis_index("s")
    wid = s * NC + c
    _zero_rows128(cnt_v, N_PAD // 128)
    ones = jnp.ones((16,), _f32)

    def body(k, _):
        base = (wid * CPT_DEG + k * SUP) * CH
        pltpu.sync_copy(dst_ref.at[pl.ds(base, SUP * CH)], idx_v)
        for q in range(SUP):
            for j in range(CH // 16):
                iv = idx_v[pl.ds(q * CH + j * 16, 16)]
                plsc.addupdate_scatter(
                    cnt_v, [lax.shift_right_logical(iv, 7),
                            lax.bitwise_and(iv, 127)], ones)
        return 0

    lax.fori_loop(0, CPT_DEG // SUP, body, 0)
    pltpu.sync_copy(cnt_v, out_ref.at[wid])


@functools.partial(
    pl.kernel,
    out_type=(
        jax.ShapeDtypeStruct((N_PAD, FH), _f32),
        jax.ShapeDtypeStruct((N_PAD, FH), _f32),
    ),
    mesh=_mesh,
    compiler_params=pltpu.CompilerParams(needs_layout_passes=False, use_tc_tiling_on_sc=False),
    scratch_types=[
        pltpu.VMEM((RING, SUP * CH), jnp.int32),
        pltpu.VMEM((RING, SUP, CH), jnp.int32),
        pltpu.VMEM((RING, CH, FH), _f32),
        pltpu.VMEM_SHARED((N_PAD, FH), _f32),
        [pltpu.SemaphoreType.DMA] * RING,
        [pltpu.SemaphoreType.DMA] * RING,
        [pltpu.SemaphoreType.DMA] * RING,
    ],
)
def _sc_agg(g0_ref, g1_ref, src_ref, dst_ref, a0_ref, a1_ref,
            src_v, dst_v, rows_v, acc_sh, sem_i, sem_g, sem_s):
    c = lax.axis_index("c")
    s = lax.axis_index("s")

    def zrow(i, _):
        rows_v[0, i, pl.ds(0, 16)] = _zeros16()
        rows_v[0, i, pl.ds(16, 16)] = _zeros16()
        return 0

    lax.fori_loop(0, CH, zrow, 0, unroll=2)

    def zacc(k, _):
        pltpu.sync_copy(rows_v.at[0], acc_sh.at[pl.ds(s * NPT + k * CH, CH)])
        return 0

    lax.fori_loop(0, NPT // CH, zacc, 0)
    plsc.subcore_barrier()

    cbase = s * CPT_AGG

    def fire_idx(sup, q):
        ebase = (cbase + sup * SUP) * CH
        pltpu.async_copy(
            src_ref.at[pl.ds(ebase, SUP * CH)], src_v.at[q], sem_i[q])
        for j in range(SUP):
            pltpu.async_copy(
                dst_ref.at[pl.ds(ebase + j * CH, CH)], dst_v.at[q, j],
                sem_i[q])

    def wait_idx(q):
        pltpu.make_async_copy(
            src_ref.at[pl.ds(0, SUP * CH)], src_v.at[q], sem_i[q]).wait()
        for j in range(SUP):
            pltpu.make_async_copy(
                dst_ref.at[pl.ds(0, CH)], dst_v.at[q, j], sem_i[q]).wait()

    def drain_scatter(r):
        pltpu.make_async_copy(
            rows_v.at[r], acc_sh.at[dst_v.at[0, 0]], sem_s[r]).wait()

    def wait_gather(r):
        pltpu.make_async_copy(
            g0_ref.at[src_v.at[0, pl.ds(0, CH)]], rows_v.at[r],
            sem_g[r]).wait()

    def fire_gather(q, j, r):
        sl = src_v.at[q, pl.ds(j * CH, CH)]

        def g0():
            pltpu.async_copy(g0_ref.at[sl], rows_v.at[r], sem_g[r])

        def g1():
            pltpu.async_copy(g1_ref.at[sl], rows_v.at[r], sem_g[r])

        pl.when(c == 0)(g0)
        pl.when(c == 1)(g1)

    def fire_scatter(q, j, r):
        pltpu.async_copy(
            rows_v.at[r], acc_sh.at[dst_v.at[q, j]], sem_s[r], add=True)

    GROUP = SUP * RING

    fire_idx(0, 0)
    fire_idx(1, 1)

    def outer(i, _):
        t0 = i * GROUP
        for tt in range(GROUP):
            jj = tt % SUP
            q = (tt // SUP) % RING
            r = tt % RING
            if jj == 0:
                S = i * RING + tt // SUP
                wait_idx(q)
                pl.when(S + 2 < NSUPER)(
                    lambda S=S, q=q: fire_idx(S + 2, (q + 2) % RING))
            t = t0 + tt
            pl.when(t >= RING)(lambda r=r: drain_scatter(r))
            fire_gather(q, jj, r)
            tb = (tt - 3) % GROUP
            qb = (tb // SUP) % RING
            jb = tb % SUP
            rb = tb % RING

            def consume(qb=qb, jb=jb, rb=rb):
                wait_gather(rb)
                fire_scatter(qb, jb, rb)

            pl.when(t >= 3)(consume)
        return 0

    lax.fori_loop(0, NSUPER // RING, outer, 0)
    TOT = CPT_AGG
    for u in (TOT - 3, TOT - 2, TOT - 1):
        qb = (u // SUP) % RING
        jb = u % SUP
        rb = u % RING
        wait_gather(rb)
        fire_scatter(qb, jb, rb)
    for u in range(TOT - RING, TOT):
        drain_scatter(u % RING)
    plsc.subcore_barrier()
    sl = pl.ds(s * NPT, NPT)
    pl.when(c == 0)(lambda: pltpu.sync_copy(acc_sh.at[sl], a0_ref.at[sl]))
    pl.when(c == 1)(lambda: pltpu.sync_copy(acc_sh.at[sl], a1_ref.at[sl]))


@functools.partial(
    pl.kernel,
    out_type=(
        jax.ShapeDtypeStruct((G_PAD, FH), _f32),
        jax.ShapeDtypeStruct((G_PAD, FH), _f32),
        jax.ShapeDtypeStruct((NS, G_PAD // 128, 128), _f32),
    ),
    mesh=_mesh,
    compiler_params=pltpu.CompilerParams(needs_layout_passes=False, use_tc_tiling_on_sc=False),
    scratch_types=[
        pltpu.VMEM((CH,), jnp.int32),
        pltpu.VMEM((CH, FH), _f32),
        pltpu.VMEM((CH, FH), _f32),
        pltpu.VMEM((G_PAD // 128, 128), _f32),
        pltpu.VMEM_SHARED((G_PAD, FH), _f32),
    ],
)
def _sc_pool(h0_ref, h1_ref, batch_ref, p0_ref, p1_ref, cnt_ref,
             idx_v, rows_v, zbuf_v, cnt_v, acc_sh):
    c = lax.axis_index("c")
    s = lax.axis_index("s")
    _zero_rows32(zbuf_v, CH)
    _zero_rows128(cnt_v, G_PAD // 128)
    pltpu.sync_copy(zbuf_v.at[pl.ds(0, GPT)], acc_sh.at[pl.ds(s * GPT, GPT)])
    plsc.subcore_barrier()
    ones = jnp.ones((16,), _f32)

    def body(k, _):
        base = s * NPT + k * CH
        pltpu.sync_copy(batch_ref.at[pl.ds(base, CH)], idx_v)
        pl.when(c == 0)(
            lambda: pltpu.sync_copy(h0_ref.at[pl.ds(base, CH)], rows_v))
        pl.when(c == 1)(
            lambda: pltpu.sync_copy(h1_ref.at[pl.ds(base, CH)], rows_v))
        pltpu.sync_copy(rows_v, acc_sh.at[idx_v], add=True)

        def count():
            for j in range(CH // 16):
                iv = idx_v[pl.ds(j * 16, 16)]
                plsc.addupdate_scatter(
                    cnt_v, [lax.shift_right_logical(iv, 7),
                            lax.bitwise_and(iv, 127)], ones)
        pl.when(c == 0)(count)
        return 0

    lax.fori_loop(0, NPT // CH, body, 0)
    plsc.subcore_barrier()
    sl = pl.ds(s * GPT, GPT)
    pl.when(c == 0)(lambda: pltpu.sync_copy(acc_sh.at[sl], p0_ref.at[sl]))
    pl.when(c == 1)(lambda: pltpu.sync_copy(acc_sh.at[sl], p1_ref.at[sl]))
    pl.when(c == 0)(lambda: pltpu.sync_copy(cnt_v, cnt_ref.at[s]))


_BN = 1024


def _tc_a_body(x_ref, degp_ref, w1_ref, g0_ref, g1_ref, dinv_ref):
    deg = jnp.sum(degp_ref[...], axis=0)
    dinv = lax.rsqrt(deg + 1.0)
    hw = jnp.dot(x_ref[...], w1_ref[...], preferred_element_type=_f32)
    g = hw * dinv[:, None]
    g0_ref[...] = g[:, :FH]
    g1_ref[...] = g[:, FH:]
    dinv_ref[...] = dinv[:, None]


def _tc_a(xp, degp, W1):
    grid = (N_PAD // _BN,)
    return pl.pallas_call(
        _tc_a_body,
        grid=grid,
        in_specs=[
            pl.BlockSpec((_BN, F), lambda i: (i, 0)),
            pl.BlockSpec((NC * NS, _BN), lambda i: (0, i)),
            pl.BlockSpec((F, F), lambda i: (0, 0)),
        ],
        out_specs=(
            pl.BlockSpec((_BN, FH), lambda i: (i, 0)),
            pl.BlockSpec((_BN, FH), lambda i: (i, 0)),
            pl.BlockSpec((_BN, 1), lambda i: (i, 0)),
        ),
        out_shape=(
            jax.ShapeDtypeStruct((N_PAD, FH), _f32),
            jax.ShapeDtypeStruct((N_PAD, FH), _f32),
            jax.ShapeDtypeStruct((N_PAD, 1), _f32),
        ),
    )(xp, degp, W1)


def _tc_mid_body(a0_ref, a1_ref, g0_ref, g1_ref, dinv_ref, w_ref, b_ref,
                 o0_ref, o1_ref):
    dinv = dinv_ref[...]
    hfull = jnp.concatenate(
        [a0_ref[...] + g0_ref[...], a1_ref[...] + g1_ref[...]], axis=1)
    h = jnp.maximum(hfull * dinv + b_ref[...], 0.0)
    hw = jnp.dot(h, w_ref[...], preferred_element_type=_f32)
    g = hw * dinv
    o0_ref[...] = g[:, :FH]
    o1_ref[...] = g[:, FH:]


def _tc_mid(a0, a1, g0, g1, dinv, W2, b1):
    grid = (N_PAD // _BN,)
    nspec = pl.BlockSpec((_BN, FH), lambda i: (i, 0))
    return pl.pallas_call(
        _tc_mid_body,
        grid=grid,
        in_specs=[
            nspec, nspec, nspec, nspec,
            pl.BlockSpec((_BN, 1), lambda i: (i, 0)),
            pl.BlockSpec((F, F), lambda i: (0, 0)),
            pl.BlockSpec((1, F), lambda i: (0, 0)),
        ],
        out_specs=(nspec, nspec),
        out_shape=(
            jax.ShapeDtypeStruct((N_PAD, FH), _f32),
            jax.ShapeDtypeStruct((N_PAD, FH), _f32),
        ),
    )(a0, a1, g0, g1, dinv, W2, b1)


def _tc_last_body(a0_ref, a1_ref, g0_ref, g1_ref, dinv_ref, b_ref,
                  o0_ref, o1_ref):
    dinv = dinv_ref[...]
    b = b_ref[...]
    o0_ref[...] = jnp.maximum(
        (a0_ref[...] + g0_ref[...]) * dinv + b[:, :FH], 0.0)
    o1_ref[...] = jnp.maximum(
        (a1_ref[...] + g1_ref[...]) * dinv + b[:, FH:], 0.0)


def _tc_last(a0, a1, g0, g1, dinv, b2):
    grid = (N_PAD // _BN,)
    nspec = pl.BlockSpec((_BN, FH), lambda i: (i, 0))
    return pl.pallas_call(
        _tc_last_body,
        grid=grid,
        in_specs=[
            nspec, nspec, nspec, nspec,
            pl.BlockSpec((_BN, 1), lambda i: (i, 0)),
            pl.BlockSpec((1, F), lambda i: (0, 0)),
        ],
        out_specs=(nspec, nspec),
        out_shape=(
            jax.ShapeDtypeStruct((N_PAD, FH), _f32),
            jax.ShapeDtypeStruct((N_PAD, FH), _f32),
        ),
    )(a0, a1, g0, g1, dinv, b2)


def _tc_head_body(p0_ref, p1_ref, cntp_ref, lig_ref, add_ref, bas_ref,
                  ary_ref, el_ref, ea_ref, eb_ref, ey_ref, w1_ref, b1_ref,
                  w2_ref, b2_ref, out_ref):
    cnt = jnp.sum(cntp_ref[...], axis=0)[:G]
    psum = jnp.concatenate([p0_ref[...], p1_ref[...]], axis=1)[:G]
    pooled = psum / jnp.maximum(cnt, 1.0)[:, None]

    w1 = w1_ref[...]
    z = jnp.dot(pooled, w1[:F], preferred_element_type=_f32)

    def emb(idx_ref, table_ref, row0, nrows):
        k = table_ref.shape[0]
        oh = (idx_ref[...] ==
              lax.broadcasted_iota(jnp.int32, (G, k), 1)).astype(_f32)
        tw = jnp.dot(table_ref[...], w1[row0:row0 + nrows],
                     preferred_element_type=_f32)
        return jnp.dot(oh, tw, preferred_element_type=_f32)

    EMB = 16
    z = z + emb(lig_ref, el_ref, F, EMB)
    z = z + emb(add_ref, ea_ref, F + EMB, EMB)
    z = z + emb(bas_ref, eb_ref, F + 2 * EMB, EMB)
    z = z + emb(ary_ref, ey_ref, F + 3 * EMB, EMB)
    z = jnp.maximum(z + b1_ref[...], 0.0)
    out_ref[...] = (jnp.dot(z, w2_ref[...], preferred_element_type=_f32)
                    + b2_ref[...])


def _tc_head(p0, p1, cntp, lig, add, bas, ary, E_lig, E_add, E_base, E_aryl,
             lin1_W, lin1_b, lin2_W, lin2_b):
    args = (p0, p1, cntp, lig, add, bas, ary, E_lig, E_add, E_base, E_aryl,
            lin1_W, lin1_b, lin2_W, lin2_b)

    def spec(a):
        nd = a.ndim
        return pl.BlockSpec(a.shape, lambda: (0,) * nd)

    return pl.pallas_call(
        _tc_head_body,
        in_specs=[spec(a) for a in args],
        out_specs=pl.BlockSpec((G, 1), lambda: (0, 0)),
        out_shape=jax.ShapeDtypeStruct((G, 1), _f32),
    )(*args)


def kernel(x, edge_index, batch, ligand_idx, additive_idx, base_idx, aryl_idx,
           W1, b1, W2, b2, E_lig, E_add, E_base, E_aryl,
           lin1_W, lin1_b, lin2_W, lin2_b):
    xp = jnp.pad(x, ((0, N_PAD - N), (0, 0)))
    ep = jnp.pad(edge_index, ((0, 0), (0, E_PAD - E)),
                 constant_values=N_PAD - 1)
    srcp = ep[0]
    dstp = ep[1]
    batchp = jnp.pad(batch, (0, N_PAD - N), constant_values=G)

    degp = _sc_deg(dstp).reshape(NC * NS, N_PAD)
    g0, g1, dinv = _tc_a(xp, degp, W1)
    a0, a1 = _sc_agg(g0, g1, srcp, dstp)
    g20, g21 = _tc_mid(a0, a1, g0, g1, dinv, W2, b1.reshape(1, F))
    a20, a21 = _sc_agg(g20, g21, srcp, dstp)
    h0, h1 = _tc_last(a20, a21, g20, g21, dinv, b2.reshape(1, F))
    p0, p1, cntp = _sc_pool(h0, h1, batchp)
    out = _tc_head(
        p0, p1, cntp.reshape(NS, G_PAD),
        ligand_idx.reshape(G, 1), additive_idx.reshape(G, 1),
        base_idx.reshape(G, 1), aryl_idx.reshape(G, 1),
        E_lig, E_add, E_base, E_aryl,
        lin1_W, lin1_b.reshape(1, F), lin2_W, lin2_b.reshape(1, 1))
    return out


# speedup vs baseline: 33.8553x; 1.6861x over previous
import functools

import jax
import jax.numpy as jnp
from jax import lax
from jax.experimental import pallas as pl
from jax.experimental.pallas import tpu as pltpu
from jax.experimental.pallas import tpu_sc as plsc

N = 50000
E = 800000
G = 512
F = 64
FH = 32
NC = 2
NS = 16
CH = 128

N_PAD = 51200
E_PAD = 819200
G_PAD = 640
NPT = N_PAD // NS
NCHUNK = E_PAD // CH
SUP = 4
RING = 5
CPT_AGG = NCHUNK // NS
NSUPER = CPT_AGG // SUP
CPT_DEG = NCHUNK // (NC * NS)
GPT = G_PAD // NS

_mesh = plsc.VectorSubcoreMesh(
    core_axis_name="c", subcore_axis_name="s", num_cores=NC, num_subcores=NS)

_f32 = jnp.float32
_zeros16 = functools.partial(jnp.zeros, (16,), _f32)


def _zero_rows32(ref, nrows):
    def body(i, _):
        ref[i, pl.ds(0, 16)] = _zeros16()
        ref[i, pl.ds(16, 16)] = _zeros16()
        return 0
    lax.fori_loop(0, nrows, body, 0, unroll=2)


def _zero_rows128(ref, nrows):
    def body(i, _):
        for j in range(8):
            ref[i, pl.ds(j * 16, 16)] = _zeros16()
        return 0
    lax.fori_loop(0, nrows, body, 0)


_DROWS = N_PAD // 128


@functools.partial(
    pl.kernel,
    out_type=jax.ShapeDtypeStruct((NC, N_PAD // 128, 128), _f32),
    mesh=_mesh,
    compiler_params=pltpu.CompilerParams(needs_layout_passes=False, use_tc_tiling_on_sc=False),
    scratch_types=[
        pltpu.VMEM((2, SUP * CH), jnp.int32),
        pltpu.VMEM((N_PAD // 128, 128), _f32),
        pltpu.VMEM((_DROWS,), jnp.int32),
        pltpu.VMEM_SHARED((N_PAD // 128, 128), _f32),
        pltpu.SemaphoreType.DMA,
        pltpu.SemaphoreType.DMA,
        pltpu.SemaphoreType.DMA,
    ],
)
def _sc_deg(dst_ref, out_ref, idx_v, cnt_v, rows_v, acc_sh, sem0, sem1, sem2):
    c = lax.axis_index("c")
    s = lax.axis_index("s")
    wid = s * NC + c
    sem = (sem0, sem1)
    _zero_rows128(cnt_v, N_PAD // 128)
    pltpu.sync_copy(cnt_v.at[pl.ds(0, _DROWS // NS)],
                    acc_sh.at[pl.ds(s * (_DROWS // NS), _DROWS // NS)])
    plsc.subcore_barrier()
    ones = jnp.ones((16,), _f32)
    ebase0 = wid * CPT_DEG * CH

    def fire(k, p):
        pltpu.async_copy(
            dst_ref.at[pl.ds(ebase0 + k * SUP * CH, SUP * CH)], idx_v.at[p],
            sem[p])

    def wait(p):
        pltpu.make_async_copy(
            dst_ref.at[pl.ds(0, SUP * CH)], idx_v.at[p], sem[p]).wait()

    NSUP_DEG = CPT_DEG // SUP

    def half(i, p):
        k = 2 * i + p
        wait(p)
        pl.when(k + 2 < NSUP_DEG)(lambda: fire(k + 2, p))
        for q in range(SUP):
            for j in range(CH // 16):
                iv = idx_v[p, pl.ds(q * CH + j * 16, 16)]
                plsc.addupdate_scatter(
                    cnt_v, [lax.shift_right_logical(iv, 7),
                            lax.bitwise_and(iv, 127)], ones)

    fire(0, 0)
    fire(1, 1)

    def body(i, _):
        half(i, 0)
        half(i, 1)
        return 0

    lax.fori_loop(0, NSUP_DEG // 2, body, 0)
    i16 = lax.iota(jnp.int32, 16)

    def mkrows(i, _):
        rows_v[pl.ds(i * 16, 16)] = i16 + i * 16
        return 0

    lax.fori_loop(0, _DROWS // 16, mkrows, 0)
    chunks = ((0, 128), (128, 128), (256, 128), (384, 16))
    for o, ln in chunks:
        pltpu.async_copy(cnt_v.at[pl.ds(o, ln)],
                         acc_sh.at[rows_v.at[pl.ds(o, ln)]],
                         sem2, add=True)
    for o, ln in chunks:
        pltpu.make_async_copy(
            cnt_v.at[pl.ds(0, ln)],
            acc_sh.at[rows_v.at[pl.ds(0, ln)]], sem2).wait()
    plsc.subcore_barrier()
    pltpu.sync_copy(acc_sh.at[pl.ds(s * (_DROWS // NS), _DROWS // NS)],
                    out_ref.at[c, pl.ds(s * (_DROWS // NS), _DROWS // NS)])


@functools.partial(
    pl.kernel,
    out_type=(
        jax.ShapeDtypeStruct((N_PAD, FH), _f32),
        jax.ShapeDtypeStruct((N_PAD, FH), _f32),
    ),
    mesh=_mesh,
    compiler_params=pltpu.CompilerParams(needs_layout_passes=False, use_tc_tiling_on_sc=False),
    scratch_types=[
        pltpu.VMEM((RING, SUP * CH), jnp.int32),
        pltpu.VMEM((RING, SUP, CH), jnp.int32),
        pltpu.VMEM((RING, CH, FH), _f32),
        pltpu.VMEM_SHARED((N_PAD, FH), _f32),
        [pltpu.SemaphoreType.DMA] * RING,
        [pltpu.SemaphoreType.DMA] * RING,
        [pltpu.SemaphoreType.DMA] * RING,
    ],
)
def _sc_agg(g0_ref, g1_ref, src_ref, dst_ref, a0_ref, a1_ref,
            src_v, dst_v, rows_v, acc_sh, sem_i, sem_g, sem_s):
    c = lax.axis_index("c")
    s = lax.axis_index("s")

    def zrow(i, _):
        rows_v[0, i, pl.ds(0, 16)] = _zeros16()
        rows_v[0, i, pl.ds(16, 16)] = _zeros16()
        return 0

    lax.fori_loop(0, CH, zrow, 0, unroll=2)

    def zacc(k, _):
        pltpu.sync_copy(rows_v.at[0], acc_sh.at[pl.ds(s * NPT + k * CH, CH)])
        return 0

    lax.fori_loop(0, NPT // CH, zacc, 0)
    plsc.subcore_barrier()

    cbase = s * CPT_AGG

    def fire_idx(sup, q):
        ebase = (cbase + sup * SUP) * CH
        pltpu.async_copy(
            src_ref.at[pl.ds(ebase, SUP * CH)], src_v.at[q], sem_i[q])
        for j in range(SUP):
            pltpu.async_copy(
                dst_ref.at[pl.ds(ebase + j * CH, CH)], dst_v.at[q, j],
                sem_i[q])

    def wait_idx(q):
        pltpu.make_async_copy(
            src_ref.at[pl.ds(0, SUP * CH)], src_v.at[q], sem_i[q]).wait()
        for j in range(SUP):
            pltpu.make_async_copy(
                dst_ref.at[pl.ds(0, CH)], dst_v.at[q, j], sem_i[q]).wait()

    def drain_scatter(r):
        pltpu.make_async_copy(
            rows_v.at[r], acc_sh.at[dst_v.at[0, 0]], sem_s[r]).wait()

    def wait_gather(r):
        pltpu.make_async_copy(
            g0_ref.at[src_v.at[0, pl.ds(0, CH)]], rows_v.at[r],
            sem_g[r]).wait()

    def fire_gather(q, j, r):
        sl = src_v.at[q, pl.ds(j * CH, CH)]

        def g0():
            pltpu.async_copy(g0_ref.at[sl], rows_v.at[r], sem_g[r])

        def g1():
            pltpu.async_copy(g1_ref.at[sl], rows_v.at[r], sem_g[r])

        pl.when(c == 0)(g0)
        pl.when(c == 1)(g1)

    def fire_scatter(q, j, r):
        pltpu.async_copy(
            rows_v.at[r], acc_sh.at[dst_v.at[q, j]], sem_s[r], add=True)

    GROUP = SUP * RING

    fire_idx(0, 0)
    fire_idx(1, 1)

    def outer(i, _):
        t0 = i * GROUP
        for tt in range(GROUP):
            jj = tt % SUP
            q = (tt // SUP) % RING
            r = tt % RING
            if jj == 0:
                S = i * RING + tt // SUP
                wait_idx(q)
                pl.when(S + 2 < NSUPER)(
                    lambda S=S, q=q: fire_idx(S + 2, (q + 2) % RING))
            t = t0 + tt
            pl.when(t >= RING)(lambda r=r: drain_scatter(r))
            fire_gather(q, jj, r)
            tb = (tt - 3) % GROUP
            qb = (tb // SUP) % RING
            jb = tb % SUP
            rb = tb % RING

            def consume(qb=qb, jb=jb, rb=rb):
                wait_gather(rb)
                fire_scatter(qb, jb, rb)

            pl.when(t >= 3)(consume)
        return 0

    lax.fori_loop(0, NSUPER // RING, outer, 0)
    TOT = CPT_AGG
    for u in (TOT - 3, TOT - 2, TOT - 1):
        qb = (u // SUP) % RING
        jb = u % SUP
        rb = u % RING
        wait_gather(rb)
        fire_scatter(qb, jb, rb)
    for u in range(TOT - RING, TOT):
        drain_scatter(u % RING)
    plsc.subcore_barrier()
    sl = pl.ds(s * NPT, NPT)
    pl.when(c == 0)(lambda: pltpu.sync_copy(acc_sh.at[sl], a0_ref.at[sl]))
    pl.when(c == 1)(lambda: pltpu.sync_copy(acc_sh.at[sl], a1_ref.at[sl]))


@functools.partial(
    pl.kernel,
    out_type=(
        jax.ShapeDtypeStruct((G_PAD, FH), _f32),
        jax.ShapeDtypeStruct((G_PAD, FH), _f32),
        jax.ShapeDtypeStruct((NS, G_PAD // 128, 128), _f32),
    ),
    mesh=_mesh,
    compiler_params=pltpu.CompilerParams(needs_layout_passes=False, use_tc_tiling_on_sc=False),
    scratch_types=[
        pltpu.VMEM((CH,), jnp.int32),
        pltpu.VMEM((CH, FH), _f32),
        pltpu.VMEM((CH, FH), _f32),
        pltpu.VMEM((G_PAD // 128, 128), _f32),
        pltpu.VMEM_SHARED((G_PAD, FH), _f32),
    ],
)
def _sc_pool(h0_ref, h1_ref, batch_ref, p0_ref, p1_ref, cnt_ref,
             idx_v, rows_v, zbuf_v, cnt_v, acc_sh):
    c = lax.axis_index("c")
    s = lax.axis_index("s")
    _zero_rows32(zbuf_v, CH)
    _zero_rows128(cnt_v, G_PAD // 128)
    pltpu.sync_copy(zbuf_v.at[pl.ds(0, GPT)], acc_sh.at[pl.ds(s * GPT, GPT)])
    plsc.subcore_barrier()
    ones = jnp.ones((16,), _f32)

    def body(k, _):
        base = s * NPT + k * CH
        pltpu.sync_copy(batch_ref.at[pl.ds(base, CH)], idx_v)
        pl.when(c == 0)(
            lambda: pltpu.sync_copy(h0_ref.at[pl.ds(base, CH)], rows_v))
        pl.when(c == 1)(
            lambda: pltpu.sync_copy(h1_ref.at[pl.ds(base, CH)], rows_v))
        pltpu.sync_copy(rows_v, acc_sh.at[idx_v], add=True)

        def count():
            for j in range(CH // 16):
                iv = idx_v[pl.ds(j * 16, 16)]
                plsc.addupdate_scatter(
                    cnt_v, [lax.shift_right_logical(iv, 7),
                            lax.bitwise_and(iv, 127)], ones)
        pl.when(c == 0)(count)
        return 0

    lax.fori_loop(0, NPT // CH, body, 0)
    plsc.subcore_barrier()
    sl = pl.ds(s * GPT, GPT)
    pl.when(c == 0)(lambda: pltpu.sync_copy(acc_sh.at[sl], p0_ref.at[sl]))
    pl.when(c == 1)(lambda: pltpu.sync_copy(acc_sh.at[sl], p1_ref.at[sl]))
    pl.when(c == 0)(lambda: pltpu.sync_copy(cnt_v, cnt_ref.at[s]))


_BN = 1024


def _tc_a_body(x_ref, degp_ref, w1_ref, g0_ref, g1_ref, dinv_ref):
    deg = jnp.sum(degp_ref[...], axis=0)
    dinv = lax.rsqrt(deg + 1.0)[:, None]
    hw = jnp.dot(x_ref[...], w1_ref[...], preferred_element_type=_f32)
    g = hw * dinv
    g0_ref[...] = g[:, :FH]
    g1_ref[...] = g[:, FH:]
    dinv_ref[...] = dinv


def _tc_a(xp, degp, W1):
    grid = (N_PAD // _BN,)
    return pl.pallas_call(
        _tc_a_body,
        grid=grid,
        in_specs=[
            pl.BlockSpec((_BN, F), lambda i: (i, 0)),
            pl.BlockSpec((NC, _BN), lambda i: (0, i)),
            pl.BlockSpec((F, F), lambda i: (0, 0)),
        ],
        out_specs=(
            pl.BlockSpec((_BN, FH), lambda i: (i, 0)),
            pl.BlockSpec((_BN, FH), lambda i: (i, 0)),
            pl.BlockSpec((_BN, 1), lambda i: (i, 0)),
        ),
        out_shape=(
            jax.ShapeDtypeStruct((N_PAD, FH), _f32),
            jax.ShapeDtypeStruct((N_PAD, FH), _f32),
            jax.ShapeDtypeStruct((N_PAD, 1), _f32),
        ),
    )(xp, degp, W1)


def _tc_mid_body(a0_ref, a1_ref, g0_ref, g1_ref, dinv_ref, w_ref, b_ref,
                 o0_ref, o1_ref):
    dinv = dinv_ref[...]
    hfull = jnp.concatenate(
        [a0_ref[...] + g0_ref[...], a1_ref[...] + g1_ref[...]], axis=1)
    h = jnp.maximum(hfull * dinv + b_ref[...], 0.0)
    hw = jnp.dot(h, w_ref[...], preferred_element_type=_f32)
    g = hw * dinv
    o0_ref[...] = g[:, :FH]
    o1_ref[...] = g[:, FH:]


def _tc_mid(a0, a1, g0, g1, dinv, W2, b1):
    grid = (N_PAD // _BN,)
    nspec = pl.BlockSpec((_BN, FH), lambda i: (i, 0))
    return pl.pallas_call(
        _tc_mid_body,
        grid=grid,
        in_specs=[
            nspec, nspec, nspec, nspec,
            pl.BlockSpec((_BN, 1), lambda i: (i, 0)),
            pl.BlockSpec((F, F), lambda i: (0, 0)),
            pl.BlockSpec((1, F), lambda i: (0, 0)),
        ],
        out_specs=(nspec, nspec),
        out_shape=(
            jax.ShapeDtypeStruct((N_PAD, FH), _f32),
            jax.ShapeDtypeStruct((N_PAD, FH), _f32),
        ),
    )(a0, a1, g0, g1, dinv, W2, b1)


def _tc_last_body(a0_ref, a1_ref, g0_ref, g1_ref, dinv_ref, b_ref,
                  o0_ref, o1_ref):
    dinv = dinv_ref[...]
    b = b_ref[...]
    o0_ref[...] = jnp.maximum(
        (a0_ref[...] + g0_ref[...]) * dinv + b[:, :FH], 0.0)
    o1_ref[...] = jnp.maximum(
        (a1_ref[...] + g1_ref[...]) * dinv + b[:, FH:], 0.0)


def _tc_last(a0, a1, g0, g1, dinv, b2):
    grid = (N_PAD // _BN,)
    nspec = pl.BlockSpec((_BN, FH), lambda i: (i, 0))
    return pl.pallas_call(
        _tc_last_body,
        grid=grid,
        in_specs=[
            nspec, nspec, nspec, nspec,
            pl.BlockSpec((_BN, 1), lambda i: (i, 0)),
            pl.BlockSpec((1, F), lambda i: (0, 0)),
        ],
        out_specs=(nspec, nspec),
        out_shape=(
            jax.ShapeDtypeStruct((N_PAD, FH), _f32),
            jax.ShapeDtypeStruct((N_PAD, FH), _f32),
        ),
    )(a0, a1, g0, g1, dinv, b2)


def _tc_head_body(p0_ref, p1_ref, cntp_ref, lig_ref, add_ref, bas_ref,
                  ary_ref, el_ref, ea_ref, eb_ref, ey_ref, w1_ref, b1_ref,
                  w2_ref, b2_ref, out_ref):
    cnt = jnp.sum(cntp_ref[...], axis=0)[:G]
    psum = jnp.concatenate([p0_ref[...], p1_ref[...]], axis=1)[:G]
    pooled = psum / jnp.maximum(cnt, 1.0)[:, None]

    w1 = w1_ref[...]
    z = jnp.dot(pooled, w1[:F], preferred_element_type=_f32)

    def emb(idx_ref, table_ref, row0, nrows):
        k = table_ref.shape[0]
        oh = (idx_ref[...] ==
              lax.broadcasted_iota(jnp.int32, (G, k), 1)).astype(_f32)
        tw = jnp.dot(table_ref[...], w1[row0:row0 + nrows],
                     preferred_element_type=_f32)
        return jnp.dot(oh, tw, preferred_element_type=_f32)

    EMB = 16
    z = z + emb(lig_ref, el_ref, F, EMB)
    z = z + emb(add_ref, ea_ref, F + EMB, EMB)
    z = z + emb(bas_ref, eb_ref, F + 2 * EMB, EMB)
    z = z + emb(ary_ref, ey_ref, F + 3 * EMB, EMB)
    z = jnp.maximum(z + b1_ref[...], 0.0)
    out_ref[...] = (jnp.dot(z, w2_ref[...], preferred_element_type=_f32)
                    + b2_ref[...])


def _tc_head(p0, p1, cntp, lig, add, bas, ary, E_lig, E_add, E_base, E_aryl,
             lin1_W, lin1_b, lin2_W, lin2_b):
    args = (p0, p1, cntp, lig, add, bas, ary, E_lig, E_add, E_base, E_aryl,
            lin1_W, lin1_b, lin2_W, lin2_b)

    def spec(a):
        nd = a.ndim
        return pl.BlockSpec(a.shape, lambda: (0,) * nd)

    return pl.pallas_call(
        _tc_head_body,
        in_specs=[spec(a) for a in args],
        out_specs=pl.BlockSpec((G, 1), lambda: (0, 0)),
        out_shape=jax.ShapeDtypeStruct((G, 1), _f32),
    )(*args)


def kernel(x, edge_index, batch, ligand_idx, additive_idx, base_idx, aryl_idx,
           W1, b1, W2, b2, E_lig, E_add, E_base, E_aryl,
           lin1_W, lin1_b, lin2_W, lin2_b):
    xp = jnp.pad(x, ((0, N_PAD - N), (0, 0)))
    epad = N + jnp.arange(E_PAD - E, dtype=jnp.int32) % (N_PAD - N)
    srcp = jnp.concatenate([edge_index[0], epad])
    dstp = jnp.concatenate([edge_index[1], epad])
    bpad = G + jnp.arange(N_PAD - N, dtype=jnp.int32) % (G_PAD - G)
    batchp = jnp.concatenate([batch, bpad])

    degp = _sc_deg(dstp).reshape(NC, N_PAD)
    g0, g1, dinv = _tc_a(xp, degp, W1)
    a0, a1 = _sc_agg(g0, g1, srcp, dstp)
    g20, g21 = _tc_mid(a0, a1, g0, g1, dinv, W2, b1.reshape(1, F))
    a20, a21 = _sc_agg(g20, g21, srcp, dstp)
    h0, h1 = _tc_last(a20, a21, g20, g21, dinv, b2.reshape(1, F))
    p0, p1, cntp = _sc_pool(h0, h1, batchp)
    out = _tc_head(
        p0, p1, cntp.reshape(NS, G_PAD),
        ligand_idx.reshape(G, 1), additive_idx.reshape(G, 1),
        base_idx.reshape(G, 1), aryl_idx.reshape(G, 1),
        E_lig, E_add, E_base, E_aryl,
        lin1_W, lin1_b.reshape(1, F), lin2_W, lin2_b.reshape(1, 1))
    return out


# speedup vs baseline: 35.9296x; 1.0613x over previous
import functools

import jax
import jax.numpy as jnp
from jax import lax
from jax.experimental import pallas as pl
from jax.experimental.pallas import tpu as pltpu
from jax.experimental.pallas import tpu_sc as plsc

N = 50000
E = 800000
G = 512
F = 64
FH = 32
NC = 2
NS = 16
CH = 128

N_PAD = 51200
E_PAD = 819200
G_PAD = 640
NPT = N_PAD // NS
NCHUNK = E_PAD // CH
SUP = 4
RING = 5
CPT_AGG = NCHUNK // NS
NSUPER = CPT_AGG // SUP
CPT_DEG = NCHUNK // (NC * NS)
GPT = G_PAD // NS

_mesh = plsc.VectorSubcoreMesh(
    core_axis_name="c", subcore_axis_name="s", num_cores=NC, num_subcores=NS)

_f32 = jnp.float32
_zeros16 = functools.partial(jnp.zeros, (16,), _f32)


def _zero_rows32(ref, nrows):
    def body(i, _):
        ref[i, pl.ds(0, 16)] = _zeros16()
        ref[i, pl.ds(16, 16)] = _zeros16()
        return 0
    lax.fori_loop(0, nrows, body, 0, unroll=2)


def _zero_rows128(ref, nrows):
    def body(i, _):
        for j in range(8):
            ref[i, pl.ds(j * 16, 16)] = _zeros16()
        return 0
    lax.fori_loop(0, nrows, body, 0)


_DROWS = N_PAD // 128


@functools.partial(
    pl.kernel,
    out_type=jax.ShapeDtypeStruct((NC, N_PAD // 128, 128), _f32),
    mesh=_mesh,
    compiler_params=pltpu.CompilerParams(needs_layout_passes=False, use_tc_tiling_on_sc=False),
    scratch_types=[
        pltpu.VMEM((2, SUP * CH), jnp.int32),
        pltpu.VMEM((N_PAD // 128, 128), _f32),
        pltpu.VMEM((_DROWS,), jnp.int32),
        pltpu.VMEM_SHARED((N_PAD // 128, 128), _f32),
        pltpu.SemaphoreType.DMA,
        pltpu.SemaphoreType.DMA,
        pltpu.SemaphoreType.DMA,
    ],
)
def _sc_deg(dst_ref, out_ref, idx_v, cnt_v, rows_v, acc_sh, sem0, sem1, sem2):
    c = lax.axis_index("c")
    s = lax.axis_index("s")
    wid = s * NC + c
    sem = (sem0, sem1)
    _zero_rows128(cnt_v, N_PAD // 128)
    pltpu.sync_copy(cnt_v.at[pl.ds(0, _DROWS // NS)],
                    acc_sh.at[pl.ds(s * (_DROWS // NS), _DROWS // NS)])
    plsc.subcore_barrier()
    ones = jnp.ones((16,), _f32)
    ebase0 = wid * CPT_DEG * CH

    def fire(k, p):
        pltpu.async_copy(
            dst_ref.at[pl.ds(ebase0 + k * SUP * CH, SUP * CH)], idx_v.at[p],
            sem[p])

    def wait(p):
        pltpu.make_async_copy(
            dst_ref.at[pl.ds(0, SUP * CH)], idx_v.at[p], sem[p]).wait()

    NSUP_DEG = CPT_DEG // SUP

    def half(i, p):
        k = 2 * i + p
        wait(p)
        pl.when(k + 2 < NSUP_DEG)(lambda: fire(k + 2, p))
        for q in range(SUP):
            for j in range(CH // 16):
                iv = idx_v[p, pl.ds(q * CH + j * 16, 16)]
                plsc.addupdate_scatter(
                    cnt_v, [lax.shift_right_logical(iv, 7),
                            lax.bitwise_and(iv, 127)], ones)

    fire(0, 0)
    fire(1, 1)

    def body(i, _):
        half(i, 0)
        half(i, 1)
        return 0

    lax.fori_loop(0, NSUP_DEG // 2, body, 0)
    i16 = lax.iota(jnp.int32, 16)

    def mkrows(i, _):
        rows_v[pl.ds(i * 16, 16)] = i16 + i * 16
        return 0

    lax.fori_loop(0, _DROWS // 16, mkrows, 0)
    chunks = ((0, 128), (128, 128), (256, 128), (384, 16))
    for o, ln in chunks:
        pltpu.async_copy(cnt_v.at[pl.ds(o, ln)],
                         acc_sh.at[rows_v.at[pl.ds(o, ln)]],
                         sem2, add=True)
    for o, ln in chunks:
        pltpu.make_async_copy(
            cnt_v.at[pl.ds(0, ln)],
            acc_sh.at[rows_v.at[pl.ds(0, ln)]], sem2).wait()
    plsc.subcore_barrier()
    pltpu.sync_copy(acc_sh.at[pl.ds(s * (_DROWS // NS), _DROWS // NS)],
                    out_ref.at[c, pl.ds(s * (_DROWS // NS), _DROWS // NS)])


@functools.partial(
    pl.kernel,
    out_type=(
        jax.ShapeDtypeStruct((N_PAD, FH), _f32),
        jax.ShapeDtypeStruct((N_PAD, FH), _f32),
    ),
    mesh=_mesh,
    compiler_params=pltpu.CompilerParams(needs_layout_passes=False, use_tc_tiling_on_sc=False),
    scratch_types=[
        pltpu.VMEM((RING, SUP * CH), jnp.int32),
        pltpu.VMEM((RING, SUP, CH), jnp.int32),
        pltpu.VMEM((RING, CH, FH), _f32),
        pltpu.VMEM_SHARED((N_PAD, FH), _f32),
        [pltpu.SemaphoreType.DMA] * RING,
        [pltpu.SemaphoreType.DMA] * RING,
        [pltpu.SemaphoreType.DMA] * RING,
    ],
)
def _sc_agg(g0_ref, g1_ref, src_ref, dst_ref, a0_ref, a1_ref,
            src_v, dst_v, rows_v, acc_sh, sem_i, sem_g, sem_s):
    c = lax.axis_index("c")
    s = lax.axis_index("s")

    def zrow(i, _):
        rows_v[0, i, pl.ds(0, 16)] = _zeros16()
        rows_v[0, i, pl.ds(16, 16)] = _zeros16()
        return 0

    lax.fori_loop(0, CH, zrow, 0, unroll=2)

    def zacc(k, _):
        pltpu.sync_copy(rows_v.at[0], acc_sh.at[pl.ds(s * NPT + k * CH, CH)])
        return 0

    lax.fori_loop(0, NPT // CH, zacc, 0)
    plsc.subcore_barrier()

    cbase = s * CPT_AGG

    def fire_idx(sup, q):
        ebase = (cbase + sup * SUP) * CH
        pltpu.async_copy(
            src_ref.at[pl.ds(ebase, SUP * CH)], src_v.at[q], sem_i[q])
        for j in range(SUP):
            pltpu.async_copy(
                dst_ref.at[pl.ds(ebase + j * CH, CH)], dst_v.at[q, j],
                sem_i[q])

    def wait_idx(q):
        pltpu.make_async_copy(
            src_ref.at[pl.ds(0, SUP * CH)], src_v.at[q], sem_i[q]).wait()
        for j in range(SUP):
            pltpu.make_async_copy(
                dst_ref.at[pl.ds(0, CH)], dst_v.at[q, j], sem_i[q]).wait()

    def drain_scatter(r):
        pltpu.make_async_copy(
            rows_v.at[r], acc_sh.at[dst_v.at[0, 0]], sem_s[r]).wait()

    def wait_gather(r):
        pltpu.make_async_copy(
            g0_ref.at[src_v.at[0, pl.ds(0, CH)]], rows_v.at[r],
            sem_g[r]).wait()

    def fire_gather(q, j, r):
        sl = src_v.at[q, pl.ds(j * CH, CH)]

        def g0():
            pltpu.async_copy(g0_ref.at[sl], rows_v.at[r], sem_g[r])

        def g1():
            pltpu.async_copy(g1_ref.at[sl], rows_v.at[r], sem_g[r])

        pl.when(c == 0)(g0)
        pl.when(c == 1)(g1)

    def fire_scatter(q, j, r):
        pltpu.async_copy(
            rows_v.at[r], acc_sh.at[dst_v.at[q, j]], sem_s[r], add=True)

    GROUP = SUP * RING

    fire_idx(0, 0)
    fire_idx(1, 1)

    def outer(i, _):
        t0 = i * GROUP
        for tt in range(GROUP):
            jj = tt % SUP
            q = (tt // SUP) % RING
            r = tt % RING
            if jj == 0:
                S = i * RING + tt // SUP
                wait_idx(q)
                pl.when(S + 2 < NSUPER)(
                    lambda S=S, q=q: fire_idx(S + 2, (q + 2) % RING))
            t = t0 + tt
            pl.when(t >= RING)(lambda r=r: drain_scatter(r))
            fire_gather(q, jj, r)
            tb = (tt - 3) % GROUP
            qb = (tb // SUP) % RING
            jb = tb % SUP
            rb = tb % RING

            def consume(qb=qb, jb=jb, rb=rb):
                wait_gather(rb)
                fire_scatter(qb, jb, rb)

            pl.when(t >= 3)(consume)
        return 0

    lax.fori_loop(0, NSUPER // RING, outer, 0)
    TOT = CPT_AGG
    for u in (TOT - 3, TOT - 2, TOT - 1):
        qb = (u // SUP) % RING
        jb = u % SUP
        rb = u % RING
        wait_gather(rb)
        fire_scatter(qb, jb, rb)
    for u in range(TOT - RING, TOT):
        drain_scatter(u % RING)
    plsc.subcore_barrier()
    sl = pl.ds(s * NPT, NPT)
    pl.when(c == 0)(lambda: pltpu.sync_copy(acc_sh.at[sl], a0_ref.at[sl]))
    pl.when(c == 1)(lambda: pltpu.sync_copy(acc_sh.at[sl], a1_ref.at[sl]))


@functools.partial(
    pl.kernel,
    out_type=(
        jax.ShapeDtypeStruct((G_PAD, FH), _f32),
        jax.ShapeDtypeStruct((G_PAD, FH), _f32),
        jax.ShapeDtypeStruct((NS, G_PAD // 128, 128), _f32),
    ),
    mesh=_mesh,
    compiler_params=pltpu.CompilerParams(needs_layout_passes=False, use_tc_tiling_on_sc=False),
    scratch_types=[
        pltpu.VMEM((CH,), jnp.int32),
        pltpu.VMEM((CH, FH), _f32),
        pltpu.VMEM((CH, FH), _f32),
        pltpu.VMEM((CH, FH), _f32),
        pltpu.VMEM((CH,), _f32),
        pltpu.VMEM((F,), _f32),
        pltpu.VMEM((G_PAD // 128, 128), _f32),
        pltpu.VMEM_SHARED((G_PAD, FH), _f32),
    ],
)
def _sc_pool(a0_ref, a1_ref, g0_ref, g1_ref, dinv_ref, b_ref, batch_ref,
             p0_ref, p1_ref, cnt_ref,
             idx_v, a_v, g_v, rows_v, d_v, b_v, cnt_v, acc_sh):
    c = lax.axis_index("c")
    s = lax.axis_index("s")
    _zero_rows32(rows_v, CH)
    _zero_rows128(cnt_v, G_PAD // 128)
    pltpu.sync_copy(rows_v.at[pl.ds(0, GPT)], acc_sh.at[pl.ds(s * GPT, GPT)])
    plsc.subcore_barrier()
    ones = jnp.ones((16,), _f32)
    pltpu.sync_copy(b_ref, b_v)
    b_lo = b_v[pl.ds(c * FH, 16)]
    b_hi = b_v[pl.ds(c * FH + 16, 16)]

    def body(k, _):
        base = s * NPT + k * CH
        pltpu.sync_copy(batch_ref.at[pl.ds(base, CH)], idx_v)
        pl.when(c == 0)(
            lambda: pltpu.sync_copy(a0_ref.at[pl.ds(base, CH)], a_v))
        pl.when(c == 1)(
            lambda: pltpu.sync_copy(a1_ref.at[pl.ds(base, CH)], a_v))
        pl.when(c == 0)(
            lambda: pltpu.sync_copy(g0_ref.at[pl.ds(base, CH)], g_v))
        pl.when(c == 1)(
            lambda: pltpu.sync_copy(g1_ref.at[pl.ds(base, CH)], g_v))
        pltpu.sync_copy(dinv_ref.at[pl.ds(base, CH)], d_v)

        def crow(grp, _):
            dv = d_v[pl.ds(grp * 16, 16)]
            for r in range(16):
                i = grp * 16 + r
                di = dv[r]
                rows_v[i, pl.ds(0, 16)] = jnp.maximum(
                    (a_v[i, pl.ds(0, 16)] + g_v[i, pl.ds(0, 16)]) * di
                    + b_lo, 0.0)
                rows_v[i, pl.ds(16, 16)] = jnp.maximum(
                    (a_v[i, pl.ds(16, 16)] + g_v[i, pl.ds(16, 16)]) * di
                    + b_hi, 0.0)
            return 0

        lax.fori_loop(0, CH // 16, crow, 0)
        pltpu.sync_copy(rows_v, acc_sh.at[idx_v], add=True)

        def count():
            for j in range(CH // 16):
                iv = idx_v[pl.ds(j * 16, 16)]
                plsc.addupdate_scatter(
                    cnt_v, [lax.shift_right_logical(iv, 7),
                            lax.bitwise_and(iv, 127)], ones)
        pl.when(c == 0)(count)
        return 0

    lax.fori_loop(0, NPT // CH, body, 0)
    plsc.subcore_barrier()
    sl = pl.ds(s * GPT, GPT)
    pl.when(c == 0)(lambda: pltpu.sync_copy(acc_sh.at[sl], p0_ref.at[sl]))
    pl.when(c == 1)(lambda: pltpu.sync_copy(acc_sh.at[sl], p1_ref.at[sl]))
    pl.when(c == 0)(lambda: pltpu.sync_copy(cnt_v, cnt_ref.at[s]))


_BN = 1024


def _tc_a_body(x_ref, degp_ref, w1_ref, g0_ref, g1_ref, dinv_ref):
    deg = jnp.sum(degp_ref[...], axis=1)
    dinv = lax.rsqrt(deg + 1.0)[:, None]
    hw = jnp.dot(x_ref[...], w1_ref[...], preferred_element_type=_f32)
    g = hw * dinv
    g0_ref[...] = g[:, :FH]
    g1_ref[...] = g[:, FH:]
    dinv_ref[...] = dinv


_BA = 400


def _tc_a(x, degp, W1):
    grid = (N_PAD // _BA,)
    nb = N // _BA
    return pl.pallas_call(
        _tc_a_body,
        grid=grid,
        in_specs=[
            pl.BlockSpec((_BA, F), lambda i: (jnp.minimum(i, nb - 1), 0)),
            pl.BlockSpec((_BA, NC), lambda i: (i, 0)),
            pl.BlockSpec((F, F), lambda i: (0, 0)),
        ],
        out_specs=(
            pl.BlockSpec((_BA, FH), lambda i: (i, 0)),
            pl.BlockSpec((_BA, FH), lambda i: (i, 0)),
            pl.BlockSpec((_BA, 1), lambda i: (i, 0)),
        ),
        out_shape=(
            jax.ShapeDtypeStruct((N_PAD, FH), _f32),
            jax.ShapeDtypeStruct((N_PAD, FH), _f32),
            jax.ShapeDtypeStruct((N_PAD, 1), _f32),
        ),
    )(x, degp, W1)


def _tc_mid_body(a0_ref, a1_ref, g0_ref, g1_ref, dinv_ref, w_ref, b_ref,
                 o0_ref, o1_ref):
    dinv = dinv_ref[...]
    hfull = jnp.concatenate(
        [a0_ref[...] + g0_ref[...], a1_ref[...] + g1_ref[...]], axis=1)
    h = jnp.maximum(hfull * dinv + b_ref[...], 0.0)
    hw = jnp.dot(h, w_ref[...], preferred_element_type=_f32)
    g = hw * dinv
    o0_ref[...] = g[:, :FH]
    o1_ref[...] = g[:, FH:]


def _tc_mid(a0, a1, g0, g1, dinv, W2, b1):
    grid = (N_PAD // _BN,)
    nspec = pl.BlockSpec((_BN, FH), lambda i: (i, 0))
    return pl.pallas_call(
        _tc_mid_body,
        grid=grid,
        in_specs=[
            nspec, nspec, nspec, nspec,
            pl.BlockSpec((_BN, 1), lambda i: (i, 0)),
            pl.BlockSpec((F, F), lambda i: (0, 0)),
            pl.BlockSpec((1, F), lambda i: (0, 0)),
        ],
        out_specs=(nspec, nspec),
        out_shape=(
            jax.ShapeDtypeStruct((N_PAD, FH), _f32),
            jax.ShapeDtypeStruct((N_PAD, FH), _f32),
        ),
    )(a0, a1, g0, g1, dinv, W2, b1)


def _tc_head_body(p0_ref, p1_ref, cntp_ref, lig_ref, add_ref, bas_ref,
                  ary_ref, el_ref, ea_ref, eb_ref, ey_ref, w1_ref, b1_ref,
                  w2_ref, b2_ref, out_ref):
    cnt = jnp.sum(cntp_ref[...], axis=0)[:G]
    psum = jnp.concatenate([p0_ref[...], p1_ref[...]], axis=1)[:G]
    pooled = psum / jnp.maximum(cnt, 1.0)[:, None]

    w1 = w1_ref[...]
    z = jnp.dot(pooled, w1[:F], preferred_element_type=_f32)

    def emb(idx_ref, table_ref, row0, nrows):
        k = table_ref.shape[0]
        oh = (idx_ref[...] ==
              lax.broadcasted_iota(jnp.int32, (G, k), 1)).astype(_f32)
        tw = jnp.dot(table_ref[...], w1[row0:row0 + nrows],
                     preferred_element_type=_f32)
        return jnp.dot(oh, tw, preferred_element_type=_f32)

    EMB = 16
    z = z + emb(lig_ref, el_ref, F, EMB)
    z = z + emb(add_ref, ea_ref, F + EMB, EMB)
    z = z + emb(bas_ref, eb_ref, F + 2 * EMB, EMB)
    z = z + emb(ary_ref, ey_ref, F + 3 * EMB, EMB)
    z = jnp.maximum(z + b1_ref[...], 0.0)
    out_ref[...] = (jnp.dot(z, w2_ref[...], preferred_element_type=_f32)
                    + b2_ref[...])


def _tc_head(p0, p1, cntp, lig, add, bas, ary, E_lig, E_add, E_base, E_aryl,
             lin1_W, lin1_b, lin2_W, lin2_b):
    args = (p0, p1, cntp, lig, add, bas, ary, E_lig, E_add, E_base, E_aryl,
            lin1_W, lin1_b, lin2_W, lin2_b)

    def spec(a):
        nd = a.ndim
        return pl.BlockSpec(a.shape, lambda: (0,) * nd)

    return pl.pallas_call(
        _tc_head_body,
        in_specs=[spec(a) for a in args],
        out_specs=pl.BlockSpec((G, 1), lambda: (0, 0)),
        out_shape=jax.ShapeDtypeStruct((G, 1), _f32),
    )(*args)


def kernel(x, edge_index, batch, ligand_idx, additive_idx, base_idx, aryl_idx,
           W1, b1, W2, b2, E_lig, E_add, E_base, E_aryl,
           lin1_W, lin1_b, lin2_W, lin2_b):
    epad = N + jnp.arange(E_PAD - E, dtype=jnp.int32) % (N_PAD - N)
    srcp = jnp.concatenate([edge_index[0], epad])
    dstp = jnp.concatenate([edge_index[1], epad])
    bpad = G + jnp.arange(N_PAD - N, dtype=jnp.int32) % (G_PAD - G)
    batchp = jnp.concatenate([batch, bpad])

    degp = _sc_deg(dstp).reshape(NC, N_PAD).T
    g0, g1, dinv = _tc_a(x, degp, W1)
    a0, a1 = _sc_agg(g0, g1, srcp, dstp)
    g20, g21 = _tc_mid(a0, a1, g0, g1, dinv, W2, b1.reshape(1, F))
    a20, a21 = _sc_agg(g20, g21, srcp, dstp)
    p0, p1, cntp = _sc_pool(a20, a21, g20, g21, dinv.reshape(N_PAD), b2,
                            batchp)
    out = _tc_head(
        p0, p1, cntp.reshape(NS, G_PAD),
        ligand_idx.reshape(G, 1), additive_idx.reshape(G, 1),
        base_idx.reshape(G, 1), aryl_idx.reshape(G, 1),
        E_lig, E_add, E_base, E_aryl,
        lin1_W, lin1_b.reshape(1, F), lin2_W, lin2_b.reshape(1, 1))
    return out


# speedup vs baseline: 39.0732x; 1.0875x over previous
import functools

import jax
import jax.numpy as jnp
from jax import lax
from jax.experimental import pallas as pl
from jax.experimental.pallas import tpu as pltpu
from jax.experimental.pallas import tpu_sc as plsc

N = 50000
E = 800000
G = 512
F = 64
FH = 32
NC = 2
NS = 16
CH = 128

N_PAD = 51200
E_PAD = 819200
G_PAD = 640
NPT = N_PAD // NS
NCHUNK = E_PAD // CH
SUP = 4
RING = 5
CPT_AGG = NCHUNK // NS
NSUPER = CPT_AGG // SUP
CPT_DEG = NCHUNK // (NC * NS)
GPT = G_PAD // NS

_mesh = plsc.VectorSubcoreMesh(
    core_axis_name="c", subcore_axis_name="s", num_cores=NC, num_subcores=NS)

_f32 = jnp.float32
_zeros16 = functools.partial(jnp.zeros, (16,), _f32)


def _zero_rows32(ref, nrows):
    def body(i, _):
        ref[i, pl.ds(0, 16)] = _zeros16()
        ref[i, pl.ds(16, 16)] = _zeros16()
        return 0
    lax.fori_loop(0, nrows, body, 0, unroll=2)


def _zero_rows128(ref, nrows):
    def body(i, _):
        for j in range(8):
            ref[i, pl.ds(j * 16, 16)] = _zeros16()
        return 0
    lax.fori_loop(0, nrows, body, 0)


_DROWS = N_PAD // 128


@functools.partial(
    pl.kernel,
    out_type=jax.ShapeDtypeStruct((NC, N_PAD // 128, 128), _f32),
    mesh=_mesh,
    compiler_params=pltpu.CompilerParams(needs_layout_passes=False, use_tc_tiling_on_sc=False),
    scratch_types=[
        pltpu.VMEM((2, SUP * CH), jnp.int32),
        pltpu.VMEM((N_PAD // 128, 128), _f32),
        pltpu.VMEM((_DROWS,), jnp.int32),
        pltpu.VMEM_SHARED((N_PAD // 128, 128), _f32),
        pltpu.SemaphoreType.DMA,
        pltpu.SemaphoreType.DMA,
        pltpu.SemaphoreType.DMA,
    ],
)
def _sc_deg(dst_ref, out_ref, idx_v, cnt_v, rows_v, acc_sh, sem0, sem1, sem2):
    c = lax.axis_index("c")
    s = lax.axis_index("s")
    wid = s * NC + c
    sem = (sem0, sem1)
    _zero_rows128(cnt_v, N_PAD // 128)
    pltpu.sync_copy(cnt_v.at[pl.ds(0, _DROWS // NS)],
                    acc_sh.at[pl.ds(s * (_DROWS // NS), _DROWS // NS)])
    plsc.subcore_barrier()
    ones = jnp.ones((16,), _f32)
    ebase0 = wid * CPT_DEG * CH

    def fire(k, p):
        pltpu.async_copy(
            dst_ref.at[pl.ds(ebase0 + k * SUP * CH, SUP * CH)], idx_v.at[p],
            sem[p])

    def wait(p):
        pltpu.make_async_copy(
            dst_ref.at[pl.ds(0, SUP * CH)], idx_v.at[p], sem[p]).wait()

    NSUP_DEG = CPT_DEG // SUP

    def half(i, p):
        k = 2 * i + p
        wait(p)
        pl.when(k + 2 < NSUP_DEG)(lambda: fire(k + 2, p))
        for q in range(SUP):
            for j in range(CH // 16):
                iv = idx_v[p, pl.ds(q * CH + j * 16, 16)]
                plsc.addupdate_scatter(
                    cnt_v, [lax.shift_right_logical(iv, 7),
                            lax.bitwise_and(iv, 127)], ones)

    fire(0, 0)
    fire(1, 1)

    def body(i, _):
        half(i, 0)
        half(i, 1)
        return 0

    lax.fori_loop(0, NSUP_DEG // 2, body, 0)
    i16 = lax.iota(jnp.int32, 16)

    def mkrows(i, _):
        rows_v[pl.ds(i * 16, 16)] = i16 + i * 16
        return 0

    lax.fori_loop(0, _DROWS // 16, mkrows, 0)
    chunks = ((0, 128), (128, 128), (256, 128), (384, 16))
    for o, ln in chunks:
        pltpu.async_copy(cnt_v.at[pl.ds(o, ln)],
                         acc_sh.at[rows_v.at[pl.ds(o, ln)]],
                         sem2, add=True)
    for o, ln in chunks:
        pltpu.make_async_copy(
            cnt_v.at[pl.ds(0, ln)],
            acc_sh.at[rows_v.at[pl.ds(0, ln)]], sem2).wait()
    plsc.subcore_barrier()
    pltpu.sync_copy(acc_sh.at[pl.ds(s * (_DROWS // NS), _DROWS // NS)],
                    out_ref.at[c, pl.ds(s * (_DROWS // NS), _DROWS // NS)])


@functools.partial(
    pl.kernel,
    out_type=(
        jax.ShapeDtypeStruct((N_PAD, FH), _f32),
        jax.ShapeDtypeStruct((N_PAD, FH), _f32),
    ),
    mesh=_mesh,
    compiler_params=pltpu.CompilerParams(needs_layout_passes=False, use_tc_tiling_on_sc=False),
    scratch_types=[
        pltpu.VMEM((RING, SUP * CH), jnp.int32),
        pltpu.VMEM((RING, SUP, CH), jnp.int32),
        pltpu.VMEM((RING, CH, FH), _f32),
        pltpu.VMEM_SHARED((N_PAD, FH), _f32),
        [pltpu.SemaphoreType.DMA] * RING,
        [pltpu.SemaphoreType.DMA] * RING,
        [pltpu.SemaphoreType.DMA] * RING,
    ],
)
def _sc_agg(g0_ref, g1_ref, src_ref, dst_ref, a0_ref, a1_ref,
            src_v, dst_v, rows_v, acc_sh, sem_i, sem_g, sem_s):
    c = lax.axis_index("c")
    s = lax.axis_index("s")

    def zrow(i, _):
        rows_v[0, i, pl.ds(0, 16)] = _zeros16()
        rows_v[0, i, pl.ds(16, 16)] = _zeros16()
        return 0

    lax.fori_loop(0, CH, zrow, 0, unroll=2)

    def zacc(k, _):
        pltpu.sync_copy(rows_v.at[0], acc_sh.at[pl.ds(s * NPT + k * CH, CH)])
        return 0

    lax.fori_loop(0, NPT // CH, zacc, 0)
    plsc.subcore_barrier()

    cbase = s * CPT_AGG

    def fire_idx(sup, q):
        ebase = (cbase + sup * SUP) * CH
        pltpu.async_copy(
            src_ref.at[pl.ds(ebase, SUP * CH)], src_v.at[q], sem_i[q])
        for j in range(SUP):
            pltpu.async_copy(
                dst_ref.at[pl.ds(ebase + j * CH, CH)], dst_v.at[q, j],
                sem_i[q])

    def wait_idx(q):
        pltpu.make_async_copy(
            src_ref.at[pl.ds(0, SUP * CH)], src_v.at[q], sem_i[q]).wait()
        for j in range(SUP):
            pltpu.make_async_copy(
                dst_ref.at[pl.ds(0, CH)], dst_v.at[q, j], sem_i[q]).wait()

    def drain_scatter(r):
        pltpu.make_async_copy(
            rows_v.at[r], acc_sh.at[dst_v.at[0, 0]], sem_s[r]).wait()

    def wait_gather(r):
        pltpu.make_async_copy(
            g0_ref.at[src_v.at[0, pl.ds(0, CH)]], rows_v.at[r],
            sem_g[r]).wait()

    def fire_gather(q, j, r):
        sl = src_v.at[q, pl.ds(j * CH, CH)]

        def g0():
            pltpu.async_copy(g0_ref.at[sl], rows_v.at[r], sem_g[r])

        def g1():
            pltpu.async_copy(g1_ref.at[sl], rows_v.at[r], sem_g[r])

        pl.when(c == 0)(g0)
        pl.when(c == 1)(g1)

    def fire_scatter(q, j, r):
        pltpu.async_copy(
            rows_v.at[r], acc_sh.at[dst_v.at[q, j]], sem_s[r], add=True)

    GROUP = SUP * RING

    fire_idx(0, 0)
    fire_idx(1, 1)

    def outer(i, _):
        t0 = i * GROUP
        for tt in range(GROUP):
            jj = tt % SUP
            q = (tt // SUP) % RING
            r = tt % RING
            if jj == 0:
                S = i * RING + tt // SUP
                wait_idx(q)
                pl.when(S + 2 < NSUPER)(
                    lambda S=S, q=q: fire_idx(S + 2, (q + 2) % RING))
            t = t0 + tt
            pl.when(t >= RING)(lambda r=r: drain_scatter(r))
            fire_gather(q, jj, r)
            tb = (tt - 3) % GROUP
            qb = (tb // SUP) % RING
            jb = tb % SUP
            rb = tb % RING

            def consume(qb=qb, jb=jb, rb=rb):
                wait_gather(rb)
                fire_scatter(qb, jb, rb)

            pl.when(t >= 3)(consume)
        return 0

    lax.fori_loop(0, NSUPER // RING, outer, 0)
    TOT = CPT_AGG
    for u in (TOT - 3, TOT - 2, TOT - 1):
        qb = (u // SUP) % RING
        jb = u % SUP
        rb = u % RING
        wait_gather(rb)
        fire_scatter(qb, jb, rb)
    for u in range(TOT - RING, TOT):
        drain_scatter(u % RING)
    plsc.subcore_barrier()
    sl = pl.ds(s * NPT, NPT)
    pl.when(c == 0)(lambda: pltpu.sync_copy(acc_sh.at[sl], a0_ref.at[sl]))
    pl.when(c == 1)(lambda: pltpu.sync_copy(acc_sh.at[sl], a1_ref.at[sl]))


@functools.partial(
    pl.kernel,
    out_type=(
        jax.ShapeDtypeStruct((G_PAD, FH), _f32),
        jax.ShapeDtypeStruct((G_PAD, FH), _f32),
        jax.ShapeDtypeStruct((NS, G_PAD // 128, 128), _f32),
    ),
    mesh=_mesh,
    compiler_params=pltpu.CompilerParams(needs_layout_passes=False, use_tc_tiling_on_sc=False),
    scratch_types=[
        pltpu.VMEM((2, CH), jnp.int32),
        pltpu.VMEM((2, CH, FH), _f32),
        pltpu.VMEM((2, CH, FH), _f32),
        pltpu.VMEM((2, CH, FH), _f32),
        pltpu.VMEM((2, CH), _f32),
        pltpu.VMEM((F,), _f32),
        pltpu.VMEM((G_PAD // 128, 128), _f32),
        pltpu.VMEM_SHARED((G_PAD, FH), _f32),
        [pltpu.SemaphoreType.DMA] * 2,
        [pltpu.SemaphoreType.DMA] * 2,
    ],
)
def _sc_pool(a0_ref, a1_ref, g0_ref, g1_ref, dinv_ref, b_ref, batch_ref,
             p0_ref, p1_ref, cnt_ref,
             idx_v, a_v, g_v, rows_v, d_v, b_v, cnt_v, acc_sh,
             sem_in, sem_sc):
    c = lax.axis_index("c")
    s = lax.axis_index("s")
    _zero_rows32(rows_v.at[0], CH)
    _zero_rows128(cnt_v, G_PAD // 128)
    pltpu.sync_copy(rows_v.at[0, pl.ds(0, GPT)],
                    acc_sh.at[pl.ds(s * GPT, GPT)])
    plsc.subcore_barrier()
    ones = jnp.ones((16,), _f32)
    pltpu.sync_copy(b_ref, b_v)
    b_lo = b_v[pl.ds(c * FH, 16)]
    b_hi = b_v[pl.ds(c * FH + 16, 16)]
    KMAX = NPT // CH

    def fire_in(k, p):
        base = s * NPT + k * CH
        pltpu.async_copy(batch_ref.at[pl.ds(base, CH)], idx_v.at[p],
                         sem_in[p])

        def in0():
            pltpu.async_copy(a0_ref.at[pl.ds(base, CH)], a_v.at[p],
                             sem_in[p])
            pltpu.async_copy(g0_ref.at[pl.ds(base, CH)], g_v.at[p],
                             sem_in[p])

        def in1():
            pltpu.async_copy(a1_ref.at[pl.ds(base, CH)], a_v.at[p],
                             sem_in[p])
            pltpu.async_copy(g1_ref.at[pl.ds(base, CH)], g_v.at[p],
                             sem_in[p])

        pl.when(c == 0)(in0)
        pl.when(c == 1)(in1)
        pltpu.async_copy(dinv_ref.at[pl.ds(base, CH)], d_v.at[p], sem_in[p])

    def wait_in(p):
        pltpu.make_async_copy(batch_ref.at[pl.ds(0, CH)], idx_v.at[p],
                              sem_in[p]).wait()
        pltpu.make_async_copy(a0_ref.at[pl.ds(0, CH)], a_v.at[p],
                              sem_in[p]).wait()
        pltpu.make_async_copy(g0_ref.at[pl.ds(0, CH)], g_v.at[p],
                              sem_in[p]).wait()
        pltpu.make_async_copy(dinv_ref.at[pl.ds(0, CH)], d_v.at[p],
                              sem_in[p]).wait()

    def wait_sc(p):
        pltpu.make_async_copy(rows_v.at[p], acc_sh.at[idx_v.at[0]],
                              sem_sc[p]).wait()

    fire_in(0, 0)

    def half(k, p):
        pl.when(k >= 1)(lambda: wait_sc(1 - p))
        pl.when(k + 1 < KMAX)(lambda: fire_in(k + 1, 1 - p))
        wait_in(p)

        def crow(grp, _):
            dv = d_v[p, pl.ds(grp * 16, 16)]
            for r in range(16):
                i = grp * 16 + r
                di = dv[r]
                rows_v[p, i, pl.ds(0, 16)] = jnp.maximum(
                    (a_v[p, i, pl.ds(0, 16)] + g_v[p, i, pl.ds(0, 16)]) * di
                    + b_lo, 0.0)
                rows_v[p, i, pl.ds(16, 16)] = jnp.maximum(
                    (a_v[p, i, pl.ds(16, 16)] + g_v[p, i, pl.ds(16, 16)])
                    * di + b_hi, 0.0)
            return 0

        lax.fori_loop(0, CH // 16, crow, 0)
        pltpu.async_copy(rows_v.at[p], acc_sh.at[idx_v.at[p]], sem_sc[p],
                         add=True)

        def count():
            for j in range(CH // 16):
                iv = idx_v[p, pl.ds(j * 16, 16)]
                plsc.addupdate_scatter(
                    cnt_v, [lax.shift_right_logical(iv, 7),
                            lax.bitwise_and(iv, 127)], ones)
        pl.when(c == 0)(count)

    def body(i, _):
        half(2 * i, 0)
        half(2 * i + 1, 1)
        return 0

    lax.fori_loop(0, KMAX // 2, body, 0)
    half(KMAX - 1, 0)
    wait_sc(0)
    plsc.subcore_barrier()
    sl = pl.ds(s * GPT, GPT)
    pl.when(c == 0)(lambda: pltpu.sync_copy(acc_sh.at[sl], p0_ref.at[sl]))
    pl.when(c == 1)(lambda: pltpu.sync_copy(acc_sh.at[sl], p1_ref.at[sl]))
    pl.when(c == 0)(lambda: pltpu.sync_copy(cnt_v, cnt_ref.at[s]))


_BN = 1024


def _tc_a_body(x_ref, degp_ref, w1_ref, g0_ref, g1_ref, dinv_ref):
    deg = jnp.sum(degp_ref[...], axis=1)
    dinv = lax.rsqrt(deg + 1.0)[:, None]
    hw = jnp.dot(x_ref[...], w1_ref[...], preferred_element_type=_f32)
    g = hw * dinv
    g0_ref[...] = g[:, :FH]
    g1_ref[...] = g[:, FH:]
    dinv_ref[...] = dinv


_BA = 400


def _tc_a(x, degp, W1):
    grid = (N_PAD // _BA,)
    nb = N // _BA
    return pl.pallas_call(
        _tc_a_body,
        grid=grid,
        in_specs=[
            pl.BlockSpec((_BA, F), lambda i: (jnp.minimum(i, nb - 1), 0)),
            pl.BlockSpec((_BA, NC), lambda i: (i, 0)),
            pl.BlockSpec((F, F), lambda i: (0, 0)),
        ],
        out_specs=(
            pl.BlockSpec((_BA, FH), lambda i: (i, 0)),
            pl.BlockSpec((_BA, FH), lambda i: (i, 0)),
            pl.BlockSpec((_BA, 1), lambda i: (i, 0)),
        ),
        out_shape=(
            jax.ShapeDtypeStruct((N_PAD, FH), _f32),
            jax.ShapeDtypeStruct((N_PAD, FH), _f32),
            jax.ShapeDtypeStruct((N_PAD, 1), _f32),
        ),
    )(x, degp, W1)


def _tc_mid_body(a0_ref, a1_ref, g0_ref, g1_ref, dinv_ref, w_ref, b_ref,
                 o0_ref, o1_ref):
    dinv = dinv_ref[...]
    hfull = jnp.concatenate(
        [a0_ref[...] + g0_ref[...], a1_ref[...] + g1_ref[...]], axis=1)
    h = jnp.maximum(hfull * dinv + b_ref[...], 0.0)
    hw = jnp.dot(h, w_ref[...], preferred_element_type=_f32)
    g = hw * dinv
    o0_ref[...] = g[:, :FH]
    o1_ref[...] = g[:, FH:]


def _tc_mid(a0, a1, g0, g1, dinv, W2, b1):
    grid = (N_PAD // _BN,)
    nspec = pl.BlockSpec((_BN, FH), lambda i: (i, 0))
    return pl.pallas_call(
        _tc_mid_body,
        grid=grid,
        in_specs=[
            nspec, nspec, nspec, nspec,
            pl.BlockSpec((_BN, 1), lambda i: (i, 0)),
            pl.BlockSpec((F, F), lambda i: (0, 0)),
            pl.BlockSpec((1, F), lambda i: (0, 0)),
        ],
        out_specs=(nspec, nspec),
        out_shape=(
            jax.ShapeDtypeStruct((N_PAD, FH), _f32),
            jax.ShapeDtypeStruct((N_PAD, FH), _f32),
        ),
    )(a0, a1, g0, g1, dinv, W2, b1)


def _tc_head_body(p0_ref, p1_ref, cntp_ref, lig_ref, add_ref, bas_ref,
                  ary_ref, el_ref, ea_ref, eb_ref, ey_ref, w1_ref, b1_ref,
                  w2_ref, b2_ref, out_ref):
    cnt = jnp.sum(cntp_ref[...], axis=0)[:G]
    psum = jnp.concatenate([p0_ref[...], p1_ref[...]], axis=1)[:G]
    pooled = psum / jnp.maximum(cnt, 1.0)[:, None]

    w1 = w1_ref[...]
    z = jnp.dot(pooled, w1[:F], preferred_element_type=_f32)

    def emb(idx_ref, table_ref, row0, nrows):
        k = table_ref.shape[0]
        oh = (idx_ref[...] ==
              lax.broadcasted_iota(jnp.int32, (G, k), 1)).astype(_f32)
        tw = jnp.dot(table_ref[...], w1[row0:row0 + nrows],
                     preferred_element_type=_f32)
        return jnp.dot(oh, tw, preferred_element_type=_f32)

    EMB = 16
    z = z + emb(lig_ref, el_ref, F, EMB)
    z = z + emb(add_ref, ea_ref, F + EMB, EMB)
    z = z + emb(bas_ref, eb_ref, F + 2 * EMB, EMB)
    z = z + emb(ary_ref, ey_ref, F + 3 * EMB, EMB)
    z = jnp.maximum(z + b1_ref[...], 0.0)
    out_ref[...] = (jnp.dot(z, w2_ref[...], preferred_element_type=_f32)
                    + b2_ref[...])


def _tc_head(p0, p1, cntp, lig, add, bas, ary, E_lig, E_add, E_base, E_aryl,
             lin1_W, lin1_b, lin2_W, lin2_b):
    args = (p0, p1, cntp, lig, add, bas, ary, E_lig, E_add, E_base, E_aryl,
            lin1_W, lin1_b, lin2_W, lin2_b)

    def spec(a):
        nd = a.ndim
        return pl.BlockSpec(a.shape, lambda: (0,) * nd)

    return pl.pallas_call(
        _tc_head_body,
        in_specs=[spec(a) for a in args],
        out_specs=pl.BlockSpec((G, 1), lambda: (0, 0)),
        out_shape=jax.ShapeDtypeStruct((G, 1), _f32),
    )(*args)


def kernel(x, edge_index, batch, ligand_idx, additive_idx, base_idx, aryl_idx,
           W1, b1, W2, b2, E_lig, E_add, E_base, E_aryl,
           lin1_W, lin1_b, lin2_W, lin2_b):
    epad = N + jnp.arange(E_PAD - E, dtype=jnp.int32) % (N_PAD - N)
    srcp = jnp.concatenate([edge_index[0], epad])
    dstp = jnp.concatenate([edge_index[1], epad])
    bpad = G + jnp.arange(N_PAD - N, dtype=jnp.int32) % (G_PAD - G)
    batchp = jnp.concatenate([batch, bpad])

    degp = _sc_deg(dstp).reshape(NC, N_PAD).T
    g0, g1, dinv = _tc_a(x, degp, W1)
    a0, a1 = _sc_agg(g0, g1, srcp, dstp)
    g20, g21 = _tc_mid(a0, a1, g0, g1, dinv, W2, b1.reshape(1, F))
    a20, a21 = _sc_agg(g20, g21, srcp, dstp)
    p0, p1, cntp = _sc_pool(a20, a21, g20, g21, dinv.reshape(N_PAD), b2,
                            batchp)
    out = _tc_head(
        p0, p1, cntp.reshape(NS, G_PAD),
        ligand_idx.reshape(G, 1), additive_idx.reshape(G, 1),
        base_idx.reshape(G, 1), aryl_idx.reshape(G, 1),
        E_lig, E_add, E_base, E_aryl,
        lin1_W, lin1_b.reshape(1, F), lin2_W, lin2_b.reshape(1, 1))
    return out
